# Initial kernel scaffold; baseline (speedup 1.0000x reference)
import jax
import jax.numpy as jnp
from jax.experimental import pallas as pl


def kernel(node_feat, edge_attr, pos, Wn, bn, We, be, We1, be1, We2, be2, Wx1, bx1, Wx2, bx2, Wh1, bh1, Wh2, bh2, ln_g, ln_b, edge_index):
    raise NotImplementedError("write your pallas kernel here")


# speedup vs baseline: 2.0930x; 2.0930x over previous
import functools

import jax
import jax.numpy as jnp
from jax import lax
from jax.experimental import pallas as pl
from jax.experimental.pallas import tpu as pltpu
from jax.experimental.pallas import tpu_sc as plsc

N = 10000
E = 320000
D = 128
WG = 256
NRBF = 32
RMAX = 10.0
GAMMA = 1.0 / ((RMAX / NRBF) ** 2)
RES_SCALE = 1000.0

NC = 2
NS = 16
NW = NC * NS
EPW = E // NW
C = 80
NCH = EPW // C
NP = 10240
RPT = NP // NS

BE = 512
BN = 1000

_f32 = jnp.float32


def _silu(x):
    return x * jax.nn.sigmoid(x)


def _sc_gather_body(td, ts, idxd, idxs, gd, gs, idv, isv, bufd, bufs, semd, sems):
    c = lax.axis_index("c")
    s = lax.axis_index("s")
    wid = c * NS + s

    def body(i, carry):
        row = wid * NCH + i
        pltpu.sync_copy(idxd.at[row], idv)
        pltpu.sync_copy(idxs.at[row], isv)
        cp1 = pltpu.async_copy(td.at[idv], bufd, semd)
        cp2 = pltpu.async_copy(ts.at[isv], bufs, sems)
        cp1.wait()
        cp2.wait()
        pltpu.sync_copy(bufd, gd.at[pl.ds(row * C, C)])
        pltpu.sync_copy(bufs, gs.at[pl.ds(row * C, C)])
        return carry

    lax.fori_loop(0, NCH, body, 0)


_sc_gather = pl.kernel(
    _sc_gather_body,
    out_type=(
        jax.ShapeDtypeStruct((E, WG), _f32),
        jax.ShapeDtypeStruct((E, WG), _f32),
    ),
    mesh=plsc.VectorSubcoreMesh(
        core_axis_name="c", subcore_axis_name="s", num_cores=NC, num_subcores=NS
    ),
    scratch_types=[
        pltpu.VMEM((C,), jnp.int32),
        pltpu.VMEM((C,), jnp.int32),
        pltpu.VMEM((C, WG), _f32),
        pltpu.VMEM((C, WG), _f32),
        pltpu.SemaphoreType.DMA,
        pltpu.SemaphoreType.DMA,
    ],
)


def _sc_scatter_body(p, idxd, out, acc, pbuf, idv, zbuf, sem):
    c = lax.axis_index("c")
    s = lax.axis_index("s")
    wid = c * NS + s

    def zrow(r, carry):
        for j in range(D // 16):
            zbuf[r, pl.ds(j * 16, 16)] = jnp.zeros((16,), _f32)
        return carry

    lax.fori_loop(0, 32, zrow, 0)
    tbase = s * RPT

    def zc(k, carry):
        pltpu.sync_copy(zbuf, acc.at[pl.ds(tbase + k * 32, 32)])
        return carry

    lax.fori_loop(0, RPT // 32, zc, 0)
    plsc.subcore_barrier()

    def body(i, carry):
        row = wid * NCH + i
        pltpu.sync_copy(p.at[pl.ds(row * C, C)], pbuf)
        pltpu.sync_copy(idxd.at[row], idv)
        pltpu.async_copy(pbuf, acc.at[idv], sem, add=True).wait()
        return carry

    lax.fori_loop(0, NCH, body, 0)
    plsc.subcore_barrier()
    pltpu.sync_copy(acc.at[pl.ds(tbase, RPT)], out.at[pl.ds(c * NP + tbase, RPT)])


_sc_scatter = pl.kernel(
    _sc_scatter_body,
    out_type=jax.ShapeDtypeStruct((NC * NP, D), _f32),
    mesh=plsc.VectorSubcoreMesh(
        core_axis_name="c", subcore_axis_name="s", num_cores=NC, num_subcores=NS
    ),
    scratch_types=[
        pltpu.VMEM_SHARED((NP, D), _f32),
        pltpu.VMEM((C, D), _f32),
        pltpu.VMEM((C,), jnp.int32),
        pltpu.VMEM((32, D), _f32),
        pltpu.SemaphoreType.DMA,
    ],
)


def _fe_body(we, we1e, be, be1, fe, fb):
    for l in range(3):
        w1e = we1e[l]
        fe[l] = jnp.dot(we[...], w1e, preferred_element_type=_f32)
        fb[l] = jnp.dot(be[...], w1e, preferred_element_type=_f32) + be1[l][None, :]


_fe_call = pl.pallas_call(
    _fe_body,
    out_shape=(
        jax.ShapeDtypeStruct((3, NRBF, D), _f32),
        jax.ShapeDtypeStruct((3, 1, D), _f32),
    ),
)


def _init_body(nf_ref, pos_ref, wn, bn, wda, wsb, h_ref, td_ref, ts_ref):
    nf = nf_ref[...]
    nf = jnp.concatenate([nf[:, :6], nf[:, 6:7] * (1.0 / RES_SCALE)], axis=1)
    h = jnp.dot(nf, wn[...], preferred_element_type=_f32) + bn[...]
    h_ref[...] = h
    a = jnp.dot(h, wda[...], preferred_element_type=_f32)
    b = jnp.dot(h, wsb[...], preferred_element_type=_f32)
    p = pos_ref[...]
    z = jnp.zeros((BN, WG - D - 3), _f32)
    td_ref[...] = jnp.concatenate([a, p, z], axis=1)
    ts_ref[...] = jnp.concatenate([b, -p, z], axis=1)


_init_call = pl.pallas_call(
    _init_body,
    grid=(N // BN,),
    in_specs=[
        pl.BlockSpec((BN, 7), lambda i: (i, 0)),
        pl.BlockSpec((BN, 3), lambda i: (i, 0)),
        pl.BlockSpec((7, D), lambda i: (0, 0)),
        pl.BlockSpec((1, D), lambda i: (0, 0)),
        pl.BlockSpec((D, D), lambda i: (0, 0)),
        pl.BlockSpec((D, D), lambda i: (0, 0)),
    ],
    out_specs=[
        pl.BlockSpec((BN, D), lambda i: (i, 0)),
        pl.BlockSpec((BN, WG), lambda i: (i, 0)),
        pl.BlockSpec((BN, WG), lambda i: (i, 0)),
    ],
    out_shape=[
        jax.ShapeDtypeStruct((N, D), _f32),
        jax.ShapeDtypeStruct((N, WG), _f32),
        jax.ShapeDtypeStruct((N, WG), _f32),
    ],
)


def _edge_body(gd_ref, gs_ref, ea_ref, fe, fb, wd2, we2, be2, wx1, bx1, wx2t, bx2,
               p1_ref, p2_ref):
    x = gd_ref[...] + gs_ref[...]
    g = x[:, :D]
    rel = x[:, D:D + 3]
    d2 = jnp.sum(rel * rel, axis=1, keepdims=True)
    dd = ea_ref[...]
    cen = (lax.broadcasted_iota(jnp.int32, (1, NRBF), 1).astype(_f32)
           * (RMAX / (NRBF - 1)))
    rbf = jnp.exp(-GAMMA * (dd - cen) ** 2)
    pre = (g + d2 * wd2[...]
           + jnp.dot(rbf, fe[...], preferred_element_type=_f32) + fb[...])
    m = _silu(pre)
    m = _silu(jnp.dot(m, we2[...], preferred_element_type=_f32) + be2[...])
    t = _silu(jnp.dot(m, wx1[...], preferred_element_type=_f32) + bx1[...])
    w = jnp.sum(t * wx2t[...], axis=1, keepdims=True) + bx2[...]
    p1_ref[...] = m
    ones = jnp.ones((BE, 1), _f32)
    z = jnp.zeros((BE, D - 4), _f32)
    p2_ref[...] = jnp.concatenate([rel * w, ones, z], axis=1)


_edge_call = pl.pallas_call(
    _edge_body,
    grid=(E // BE,),
    in_specs=[
        pl.BlockSpec((BE, WG), lambda i: (i, 0)),
        pl.BlockSpec((BE, WG), lambda i: (i, 0)),
        pl.BlockSpec((BE, 1), lambda i: (i, 0)),
        pl.BlockSpec((NRBF, D), lambda i: (0, 0)),
        pl.BlockSpec((1, D), lambda i: (0, 0)),
        pl.BlockSpec((1, D), lambda i: (0, 0)),
        pl.BlockSpec((D, D), lambda i: (0, 0)),
        pl.BlockSpec((1, D), lambda i: (0, 0)),
        pl.BlockSpec((D, D), lambda i: (0, 0)),
        pl.BlockSpec((1, D), lambda i: (0, 0)),
        pl.BlockSpec((1, D), lambda i: (0, 0)),
        pl.BlockSpec((1, 1), lambda i: (0, 0)),
    ],
    out_specs=[
        pl.BlockSpec((BE, D), lambda i: (i, 0)),
        pl.BlockSpec((BE, D), lambda i: (i, 0)),
    ],
    out_shape=[
        jax.ShapeDtypeStruct((E, D), _f32),
        jax.ShapeDtypeStruct((E, D), _f32),
    ],
)


def _node_body(a0_ref, a1_ref, b0_ref, b1_ref, h_ref, pos_ref, wh1a, wh1b, bh1,
               wh2, bh2, lg, lb, *rest, with_tables):
    if with_tables:
        wda, wsb, hn_ref, pn_ref, td_ref, ts_ref = rest
    else:
        hn_ref, pn_ref = rest
    agg = a0_ref[...] + a1_ref[...]
    pacc = b0_ref[...] + b1_ref[...]
    posd = pacc[:, :3]
    deg = pacc[:, 3:4]
    pn = pos_ref[...] + posd / (deg + 1.0)
    hh = h_ref[...]
    u = _silu(jnp.dot(hh, wh1a[...], preferred_element_type=_f32)
              + jnp.dot(agg, wh1b[...], preferred_element_type=_f32) + bh1[...])
    hn = hh + jnp.dot(u, wh2[...], preferred_element_type=_f32) + bh2[...]
    mu = jnp.mean(hn, axis=1, keepdims=True)
    var = jnp.mean((hn - mu) ** 2, axis=1, keepdims=True)
    hn = (hn - mu) * lax.rsqrt(var + 1e-5) * lg[...] + lb[...]
    hn_ref[...] = hn
    pn_ref[...] = pn
    if with_tables:
        a = jnp.dot(hn, wda[...], preferred_element_type=_f32)
        b = jnp.dot(hn, wsb[...], preferred_element_type=_f32)
        z = jnp.zeros((BN, WG - D - 3), _f32)
        td_ref[...] = jnp.concatenate([a, pn, z], axis=1)
        ts_ref[...] = jnp.concatenate([b, -pn, z], axis=1)


def _make_node_call(with_tables):
    n_extra_in = 2 if with_tables else 0
    out_shapes = [
        jax.ShapeDtypeStruct((N, D), _f32),
        jax.ShapeDtypeStruct((N, 3), _f32),
    ]
    out_specs = [
        pl.BlockSpec((BN, D), lambda i: (i, 0)),
        pl.BlockSpec((BN, 3), lambda i: (i, 0)),
    ]
    if with_tables:
        out_shapes += [jax.ShapeDtypeStruct((N, WG), _f32)] * 2
        out_specs += [pl.BlockSpec((BN, WG), lambda i: (i, 0))] * 2
    return pl.pallas_call(
        functools.partial(_node_body, with_tables=with_tables),
        grid=(N // BN,),
        in_specs=[
            pl.BlockSpec((BN, D), lambda i: (i, 0)),
            pl.BlockSpec((BN, D), lambda i: (i, 0)),
            pl.BlockSpec((BN, D), lambda i: (i, 0)),
            pl.BlockSpec((BN, D), lambda i: (i, 0)),
            pl.BlockSpec((BN, D), lambda i: (i, 0)),
            pl.BlockSpec((BN, 3), lambda i: (i, 0)),
            pl.BlockSpec((D, D), lambda i: (0, 0)),
            pl.BlockSpec((D, D), lambda i: (0, 0)),
            pl.BlockSpec((1, D), lambda i: (0, 0)),
            pl.BlockSpec((D, D), lambda i: (0, 0)),
            pl.BlockSpec((1, D), lambda i: (0, 0)),
            pl.BlockSpec((1, D), lambda i: (0, 0)),
            pl.BlockSpec((1, D), lambda i: (0, 0)),
        ] + [pl.BlockSpec((D, D), lambda i: (0, 0))] * n_extra_in,
        out_specs=out_specs,
        out_shape=out_shapes,
    )


_node_mid = _make_node_call(True)
_node_last = _make_node_call(False)


def kernel(node_feat, edge_attr, pos, Wn, bn, We, be, We1, be1, We2, be2,
           Wx1, bx1, Wx2, bx2, Wh1, bh1, Wh2, bh2, ln_g, ln_b, edge_index):
    src = edge_index[0]
    dst = edge_index[1]
    idxd = dst.reshape(NW * NCH, C)
    idxs = src.reshape(NW * NCH, C)

    fe, fb = _fe_call(We, We1[:, 2 * D + 1:, :], be.reshape(1, D), be1)
    h, td, ts = _init_call(node_feat, pos, Wn, bn.reshape(1, D),
                           We1[0, :D, :], We1[0, D:2 * D, :])
    for l in range(3):
        gd, gs = _sc_gather(td, ts, idxd, idxs)
        p1, p2 = _edge_call(gd, gs, edge_attr, fe[l], fb[l],
                            We1[l, 2 * D, :].reshape(1, D), We2[l],
                            be2[l].reshape(1, D), Wx1[l], bx1[l].reshape(1, D),
                            Wx2[l].reshape(1, D), bx2[l].reshape(1, 1))
        agg2 = _sc_scatter(p1, idxd)
        pd2 = _sc_scatter(p2, idxd)
        if l < 2:
            h, pos, td, ts = _node_mid(
                agg2[:N], agg2[NP:NP + N], pd2[:N], pd2[NP:NP + N], h, pos,
                Wh1[l, :D, :], Wh1[l, D:, :], bh1[l].reshape(1, D),
                Wh2[l], bh2[l].reshape(1, D),
                ln_g[l].reshape(1, D), ln_b[l].reshape(1, D),
                We1[l + 1, :D, :], We1[l + 1, D:2 * D, :])
        else:
            h, pos = _node_last(
                agg2[:N], agg2[NP:NP + N], pd2[:N], pd2[NP:NP + N], h, pos,
                Wh1[l, :D, :], Wh1[l, D:, :], bh1[l].reshape(1, D),
                Wh2[l], bh2[l].reshape(1, D),
                ln_g[l].reshape(1, D), ln_b[l].reshape(1, D))
    return h, pos


# speedup vs baseline: 2.3561x; 1.1257x over previous
import functools

import jax
import jax.numpy as jnp
from jax import lax
from jax.experimental import pallas as pl
from jax.experimental.pallas import tpu as pltpu
from jax.experimental.pallas import tpu_sc as plsc

N = 10000
E = 320000
D = 128
WG = 256
WO = 144
NRBF = 32
RMAX = 10.0
GAMMA = 1.0 / ((RMAX / NRBF) ** 2)
RES_SCALE = 1000.0

NC = 2
NS = 16
NW = NC * NS
EPW = E // NW
C = 80
NCH = EPW // C
PB = 10000
NPOS = 320
NPP = 10368
RPT = NPP // NS

BE = 512
BN = 1000

_f32 = jnp.float32


def _silu(x):
    return x * jax.nn.sigmoid(x)


def _sc_gather_body(td, ts, idxd, idxs, g, idv, isv, bufd, bufs, obuf,
                    semd0, sems0, semd1, sems1):
    c = lax.axis_index("c")
    s = lax.axis_index("s")
    wid = c * NS + s
    sems = ((semd0, sems0), (semd1, sems1))

    def start(i, sl):
        row = wid * NCH + i
        pltpu.sync_copy(idxd.at[row], idv.at[sl])
        pltpu.sync_copy(idxs.at[row], isv.at[sl])
        pltpu.async_copy(td.at[idv.at[sl]], bufd.at[sl], sems[sl][0])
        pltpu.async_copy(ts.at[isv.at[sl]], bufs.at[sl], sems[sl][1])

    def finish(i, sl):
        row = wid * NCH + i
        pltpu.make_async_copy(td.at[idv.at[sl]], bufd.at[sl], sems[sl][0]).wait()
        pltpu.make_async_copy(ts.at[isv.at[sl]], bufs.at[sl], sems[sl][1]).wait()

        def add_row(r, carry):
            for j in range(WO // 16):
                obuf[sl, r, pl.ds(j * 16, 16)] = (
                    bufd[sl, r, pl.ds(j * 16, 16)] + bufs[sl, r, pl.ds(j * 16, 16)])
            return carry

        lax.fori_loop(0, C, add_row, 0)
        pltpu.sync_copy(obuf.at[sl], g.at[pl.ds(row * C, C)])

    start(0, 0)

    def body2(k, carry):
        i0 = 2 * k
        start(i0 + 1, 1)
        finish(i0, 0)
        start(i0 + 2, 0)
        finish(i0 + 1, 1)
        return carry

    lax.fori_loop(0, (NCH - 1) // 2, body2, 0)
    finish(NCH - 1, 0)


_sc_gather = pl.kernel(
    _sc_gather_body,
    out_type=jax.ShapeDtypeStruct((E, WO), _f32),
    mesh=plsc.VectorSubcoreMesh(
        core_axis_name="c", subcore_axis_name="s", num_cores=NC, num_subcores=NS
    ),
    scratch_types=[
        pltpu.VMEM((2, C), jnp.int32),
        pltpu.VMEM((2, C), jnp.int32),
        pltpu.VMEM((2, C, WG), _f32),
        pltpu.VMEM((2, C, WG), _f32),
        pltpu.VMEM((2, C, WO), _f32),
        pltpu.SemaphoreType.DMA,
        pltpu.SemaphoreType.DMA,
        pltpu.SemaphoreType.DMA,
        pltpu.SemaphoreType.DMA,
    ],
)


def _sc_scatter_body(p1, p2, idxd, idx2, out, acc, pb, qb, iv, iv2, zbuf,
                     sem0, sem1):
    c = lax.axis_index("c")
    s = lax.axis_index("s")
    wid = c * NS + s
    sems = (sem0, sem1)

    def zrow(r, carry):
        for j in range(D // 16):
            zbuf[r, pl.ds(j * 16, 16)] = jnp.zeros((16,), _f32)
        return carry

    lax.fori_loop(0, 8, zrow, 0)
    tbase = s * RPT

    def zc(k, carry):
        pltpu.sync_copy(zbuf, acc.at[pl.ds(tbase + k * 8, 8)])
        return carry

    lax.fori_loop(0, RPT // 8, zc, 0)
    plsc.subcore_barrier()

    def loads(i, sl):
        row = wid * NCH + i
        pltpu.sync_copy(p1.at[pl.ds(row * C, C)], pb.at[sl])
        pltpu.sync_copy(p2.at[pl.ds(row * C, C)], qb.at[sl])
        pltpu.sync_copy(idxd.at[row], iv.at[sl])
        pltpu.sync_copy(idx2.at[row], iv2.at[sl])

    def fire(sl):
        pltpu.async_copy(pb.at[sl], acc.at[iv.at[sl]], sems[sl], add=True)
        pltpu.async_copy(qb.at[sl], acc.at[iv2.at[sl]], sems[sl], add=True)

    def drain(sl):
        pltpu.make_async_copy(pb.at[sl], acc.at[iv.at[sl]], sems[sl]).wait()
        pltpu.make_async_copy(qb.at[sl], acc.at[iv2.at[sl]], sems[sl]).wait()

    loads(0, 0)
    loads(1, 1)

    def body2(k, carry):
        i0 = 2 * k
        fire(0)
        fire(1)
        drain(0)

        @pl.when(i0 + 2 < NCH)
        def _():
            loads(i0 + 2, 0)

        drain(1)

        @pl.when(i0 + 3 < NCH)
        def _():
            loads(i0 + 3, 1)

        return carry

    lax.fori_loop(0, NCH // 2, body2, 0)
    fire(0)
    drain(0)
    plsc.subcore_barrier()
    pltpu.sync_copy(acc.at[pl.ds(tbase, RPT)], out.at[pl.ds(c * NPP + tbase, RPT)])


_sc_scatter = pl.kernel(
    _sc_scatter_body,
    out_type=jax.ShapeDtypeStruct((NC * NPP, D), _f32),
    mesh=plsc.VectorSubcoreMesh(
        core_axis_name="c", subcore_axis_name="s", num_cores=NC, num_subcores=NS
    ),
    scratch_types=[
        pltpu.VMEM_SHARED((NPP, D), _f32),
        pltpu.VMEM((2, C, D), _f32),
        pltpu.VMEM((2, C, D), _f32),
        pltpu.VMEM((2, C), jnp.int32),
        pltpu.VMEM((2, C), jnp.int32),
        pltpu.VMEM((8, D), _f32),
        pltpu.SemaphoreType.DMA,
        pltpu.SemaphoreType.DMA,
    ],
)


def _fe_body(we, we1e, be, be1, fe, fb):
    for l in range(3):
        w1e = we1e[l]
        fe[l] = jnp.dot(we[...], w1e, preferred_element_type=_f32)
        fb[l] = jnp.dot(be[...], w1e, preferred_element_type=_f32) + be1[l][None, :]


_fe_call = pl.pallas_call(
    _fe_body,
    out_shape=(
        jax.ShapeDtypeStruct((3, NRBF, D), _f32),
        jax.ShapeDtypeStruct((3, 1, D), _f32),
    ),
)


def _init_body(nf_ref, pos_ref, wn, bn, wda, wsb, h_ref, td_ref, ts_ref):
    nf = nf_ref[...]
    nf = jnp.concatenate([nf[:, :6], nf[:, 6:7] * (1.0 / RES_SCALE)], axis=1)
    h = jnp.dot(nf, wn[...], preferred_element_type=_f32) + bn[...]
    h_ref[...] = h
    a = jnp.dot(h, wda[...], preferred_element_type=_f32)
    b = jnp.dot(h, wsb[...], preferred_element_type=_f32)
    p = pos_ref[...]
    z = jnp.zeros((BN, WG - D - 3), _f32)
    td_ref[...] = jnp.concatenate([a, p, z], axis=1)
    ts_ref[...] = jnp.concatenate([b, -p, z], axis=1)


_init_call = pl.pallas_call(
    _init_body,
    grid=(N // BN,),
    in_specs=[
        pl.BlockSpec((BN, 7), lambda i: (i, 0)),
        pl.BlockSpec((BN, 3), lambda i: (i, 0)),
        pl.BlockSpec((7, D), lambda i: (0, 0)),
        pl.BlockSpec((1, D), lambda i: (0, 0)),
        pl.BlockSpec((D, D), lambda i: (0, 0)),
        pl.BlockSpec((D, D), lambda i: (0, 0)),
    ],
    out_specs=[
        pl.BlockSpec((BN, D), lambda i: (i, 0)),
        pl.BlockSpec((BN, WG), lambda i: (i, 0)),
        pl.BlockSpec((BN, WG), lambda i: (i, 0)),
    ],
    out_shape=[
        jax.ShapeDtypeStruct((N, D), _f32),
        jax.ShapeDtypeStruct((N, WG), _f32),
        jax.ShapeDtypeStruct((N, WG), _f32),
    ],
)


def _edge_body(g_ref, ea_ref, dst_ref, fe, fb, wd2, we2, be2, wx1, bx1, wx2t,
               bx2, p1_ref, p2_ref):
    x = g_ref[...]
    gsum = x[:, :D]
    rel = x[:, D:D + 3]
    d2 = jnp.sum(rel * rel, axis=1, keepdims=True)
    dd = ea_ref[...]
    cen = (lax.broadcasted_iota(jnp.int32, (1, NRBF), 1).astype(_f32)
           * (RMAX / (NRBF - 1)))
    rbf = jnp.exp(-GAMMA * (dd - cen) ** 2)
    pre = (gsum + d2 * wd2[...]
           + jnp.dot(rbf, fe[...], preferred_element_type=_f32) + fb[...])
    m = _silu(pre)
    m = _silu(jnp.dot(m, we2[...], preferred_element_type=_f32) + be2[...])
    t = _silu(jnp.dot(m, wx1[...], preferred_element_type=_f32) + bx1[...])
    w = jnp.sum(t * wx2t[...], axis=1, keepdims=True) + bx2[...]
    p1_ref[...] = m
    rw = rel * w
    dm = lax.rem(dst_ref[...], jnp.full((BE, 1), 32, jnp.int32))
    lane = lax.broadcasted_iota(jnp.int32, (1, D), 1)
    lm = lax.rem(lane, jnp.full((1, D), 4, jnp.int32))
    grp = lax.div(lane, jnp.full((1, D), 4, jnp.int32))
    vals = (rw[:, 0:1] * (lm == 0).astype(_f32)
            + rw[:, 1:2] * (lm == 1).astype(_f32)
            + rw[:, 2:3] * (lm == 2).astype(_f32)
            + (lm == 3).astype(_f32))
    p2_ref[...] = jnp.where(grp == dm, vals, 0.0)


_edge_call = pl.pallas_call(
    _edge_body,
    grid=(E // BE,),
    in_specs=[
        pl.BlockSpec((BE, WO), lambda i: (i, 0)),
        pl.BlockSpec((BE, 1), lambda i: (i, 0)),
        pl.BlockSpec((BE, 1), lambda i: (i, 0)),
        pl.BlockSpec((NRBF, D), lambda i: (0, 0)),
        pl.BlockSpec((1, D), lambda i: (0, 0)),
        pl.BlockSpec((1, D), lambda i: (0, 0)),
        pl.BlockSpec((D, D), lambda i: (0, 0)),
        pl.BlockSpec((1, D), lambda i: (0, 0)),
        pl.BlockSpec((D, D), lambda i: (0, 0)),
        pl.BlockSpec((1, D), lambda i: (0, 0)),
        pl.BlockSpec((1, D), lambda i: (0, 0)),
        pl.BlockSpec((1, 1), lambda i: (0, 0)),
    ],
    out_specs=[
        pl.BlockSpec((BE, D), lambda i: (i, 0)),
        pl.BlockSpec((BE, D), lambda i: (i, 0)),
    ],
    out_shape=[
        jax.ShapeDtypeStruct((E, D), _f32),
        jax.ShapeDtypeStruct((E, D), _f32),
    ],
)


def _node_body(a0_ref, a1_ref, pd0_ref, pd1_ref, h_ref, pos_ref, wh1a, wh1b,
               bh1, wh2, bh2, lg, lb, *rest, with_tables):
    if with_tables:
        wda, wsb, hn_ref, pn_ref, td_ref, ts_ref = rest
    else:
        hn_ref, pn_ref = rest
    agg = a0_ref[...] + a1_ref[...]
    pacc = pd0_ref[...] + pd1_ref[...]
    posd = pacc[:, :3]
    deg = pacc[:, 3:4]
    pn = pos_ref[...] + posd / (deg + 1.0)
    hh = h_ref[...]
    u = _silu(jnp.dot(hh, wh1a[...], preferred_element_type=_f32)
              + jnp.dot(agg, wh1b[...], preferred_element_type=_f32) + bh1[...])
    hn = hh + jnp.dot(u, wh2[...], preferred_element_type=_f32) + bh2[...]
    mu = jnp.mean(hn, axis=1, keepdims=True)
    var = jnp.mean((hn - mu) ** 2, axis=1, keepdims=True)
    hn = (hn - mu) * lax.rsqrt(var + 1e-5) * lg[...] + lb[...]
    hn_ref[...] = hn
    pn_ref[...] = pn
    if with_tables:
        a = jnp.dot(hn, wda[...], preferred_element_type=_f32)
        b = jnp.dot(hn, wsb[...], preferred_element_type=_f32)
        z = jnp.zeros((BN, WG - D - 3), _f32)
        td_ref[...] = jnp.concatenate([a, pn, z], axis=1)
        ts_ref[...] = jnp.concatenate([b, -pn, z], axis=1)


def _make_node_call(with_tables):
    n_extra_in = 2 if with_tables else 0
    out_shapes = [
        jax.ShapeDtypeStruct((N, D), _f32),
        jax.ShapeDtypeStruct((N, 3), _f32),
    ]
    out_specs = [
        pl.BlockSpec((BN, D), lambda i: (i, 0)),
        pl.BlockSpec((BN, 3), lambda i: (i, 0)),
    ]
    if with_tables:
        out_shapes += [jax.ShapeDtypeStruct((N, WG), _f32)] * 2
        out_specs += [pl.BlockSpec((BN, WG), lambda i: (i, 0))] * 2
    return pl.pallas_call(
        functools.partial(_node_body, with_tables=with_tables),
        grid=(N // BN,),
        in_specs=[
            pl.BlockSpec((BN, D), lambda i: (i, 0)),
            pl.BlockSpec((BN, D), lambda i: (i, 0)),
            pl.BlockSpec((BN, 4), lambda i: (i, 0)),
            pl.BlockSpec((BN, 4), lambda i: (i, 0)),
            pl.BlockSpec((BN, D), lambda i: (i, 0)),
            pl.BlockSpec((BN, 3), lambda i: (i, 0)),
            pl.BlockSpec((D, D), lambda i: (0, 0)),
            pl.BlockSpec((D, D), lambda i: (0, 0)),
            pl.BlockSpec((1, D), lambda i: (0, 0)),
            pl.BlockSpec((D, D), lambda i: (0, 0)),
            pl.BlockSpec((1, D), lambda i: (0, 0)),
            pl.BlockSpec((1, D), lambda i: (0, 0)),
            pl.BlockSpec((1, D), lambda i: (0, 0)),
        ] + [pl.BlockSpec((D, D), lambda i: (0, 0))] * n_extra_in,
        out_specs=out_specs,
        out_shape=out_shapes,
    )


_node_mid = _make_node_call(True)
_node_last = _make_node_call(False)


def kernel(node_feat, edge_attr, pos, Wn, bn, We, be, We1, be1, We2, be2,
           Wx1, bx1, Wx2, bx2, Wh1, bh1, Wh2, bh2, ln_g, ln_b, edge_index):
    src = edge_index[0]
    dst = edge_index[1]
    idxd = dst.reshape(NW * NCH, C)
    idxs = src.reshape(NW * NCH, C)
    idx2 = (PB + dst // 32).reshape(NW * NCH, C)
    dstc = dst.reshape(E, 1)

    fe, fb = _fe_call(We, We1[:, 2 * D + 1:, :], be.reshape(1, D), be1)
    h, td, ts = _init_call(node_feat, pos, Wn, bn.reshape(1, D),
                           We1[0, :D, :], We1[0, D:2 * D, :])
    for l in range(3):
        g = _sc_gather(td, ts, idxd, idxs)
        p1, p2 = _edge_call(g, edge_attr, dstc, fe[l], fb[l],
                            We1[l, 2 * D, :].reshape(1, D), We2[l],
                            be2[l].reshape(1, D), Wx1[l], bx1[l].reshape(1, D),
                            Wx2[l].reshape(1, D), bx2[l].reshape(1, 1))
        outm = _sc_scatter(p1, p2, idxd, idx2)
        a0 = outm[:N]
        a1 = outm[NPP:NPP + N]
        pd0 = outm[PB:PB + NPOS].reshape(NPOS * 32, 4)[:N]
        pd1 = outm[NPP + PB:NPP + PB + NPOS].reshape(NPOS * 32, 4)[:N]
        if l < 2:
            h, pos, td, ts = _node_mid(
                a0, a1, pd0, pd1, h, pos,
                Wh1[l, :D, :], Wh1[l, D:, :], bh1[l].reshape(1, D),
                Wh2[l], bh2[l].reshape(1, D),
                ln_g[l].reshape(1, D), ln_b[l].reshape(1, D),
                We1[l + 1, :D, :], We1[l + 1, D:2 * D, :])
        else:
            h, pos = _node_last(
                a0, a1, pd0, pd1, h, pos,
                Wh1[l, :D, :], Wh1[l, D:, :], bh1[l].reshape(1, D),
                Wh2[l], bh2[l].reshape(1, D),
                ln_g[l].reshape(1, D), ln_b[l].reshape(1, D))
    return h, pos


# speedup vs baseline: 2.7022x; 1.1469x over previous
import functools

import jax
import jax.numpy as jnp
from jax import lax
from jax.experimental import pallas as pl
from jax.experimental.pallas import tpu as pltpu
from jax.experimental.pallas import tpu_sc as plsc

N = 10000
E = 320000
D = 128
WG = 256
WO = 144
NRBF = 32
RMAX = 10.0
GAMMA = 1.0 / ((RMAX / NRBF) ** 2)
RES_SCALE = 1000.0

NC = 2
NS = 16
NW = NC * NS
EPW = E // NW
C = 80
NCH = EPW // C
PB = 10000
NPOS = 320
NPP = 10368
RPT = NPP // NS

BE = 512
BN = 1000

_f32 = jnp.float32


def _silu(x):
    return x * jax.nn.sigmoid(x)


def _sc_gather_body(td, ts, idxd, idxs, g, idb, isb, bufd, bufs, obuf,
                    semd0, sems0, semd1, sems1, semw0):
    c = lax.axis_index("c")
    s = lax.axis_index("s")
    wid = c * NS + s
    sems = ((semd0, sems0), (semd1, sems1))

    pltpu.sync_copy(idxd.at[pl.ds(wid * EPW, EPW)], idb)
    pltpu.sync_copy(idxs.at[pl.ds(wid * EPW, EPW)], isb)

    def start(i, sl):
        pltpu.async_copy(td.at[idb.at[pl.ds(i * C, C)]], bufd.at[sl], sems[sl][0])
        pltpu.async_copy(ts.at[isb.at[pl.ds(i * C, C)]], bufs.at[sl], sems[sl][1])

    def finish(i, sl):
        row = wid * NCH + i
        pltpu.make_async_copy(td.at[idb.at[pl.ds(i * C, C)]], bufd.at[sl],
                              sems[sl][0]).wait()
        pltpu.make_async_copy(ts.at[isb.at[pl.ds(i * C, C)]], bufs.at[sl],
                              sems[sl][1]).wait()

        @pl.when(i >= 1)
        def _():
            pltpu.make_async_copy(obuf, g.at[pl.ds(row * C, C)], semw0).wait()

        def add_row(r, carry):
            for j in range(WO // 16):
                obuf[r, pl.ds(j * 16, 16)] = (
                    bufd[sl, r, pl.ds(j * 16, 16)] + bufs[sl, r, pl.ds(j * 16, 16)])
            return carry

        lax.fori_loop(0, C, add_row, 0)
        pltpu.async_copy(obuf, g.at[pl.ds(row * C, C)], semw0)

    start(0, 0)

    def body2(k, carry):
        i0 = 2 * k
        start(i0 + 1, 1)
        finish(i0, 0)
        start(i0 + 2, 0)
        finish(i0 + 1, 1)
        return carry

    lax.fori_loop(0, (NCH - 1) // 2, body2, 0)
    finish(NCH - 1, 0)
    pltpu.make_async_copy(obuf, g.at[pl.ds(0, C)], semw0).wait()


_sc_gather = pl.kernel(
    _sc_gather_body,
    out_type=jax.ShapeDtypeStruct((E, WO), _f32),
    mesh=plsc.VectorSubcoreMesh(
        core_axis_name="c", subcore_axis_name="s", num_cores=NC, num_subcores=NS
    ),
    scratch_types=[
        pltpu.VMEM((EPW,), jnp.int32),
        pltpu.VMEM((EPW,), jnp.int32),
        pltpu.VMEM((2, C, WG), _f32),
        pltpu.VMEM((2, C, WG), _f32),
        pltpu.VMEM((C, WO), _f32),
        pltpu.SemaphoreType.DMA,
        pltpu.SemaphoreType.DMA,
        pltpu.SemaphoreType.DMA,
        pltpu.SemaphoreType.DMA,
        pltpu.SemaphoreType.DMA,
    ],
)


def _sc_scatter_body(p1, p2, idxd, idx2, out, acc, pb, qb, iv, iv2, zbuf,
                     semA0, semA1, semL0, semL1):
    c = lax.axis_index("c")
    s = lax.axis_index("s")
    wid = c * NS + s
    semA = (semA0, semA1)
    semL = (semL0, semL1)

    def zrow(r, carry):
        for j in range(D // 16):
            zbuf[r, pl.ds(j * 16, 16)] = jnp.zeros((16,), _f32)
        return carry

    lax.fori_loop(0, 32, zrow, 0)
    tbase = s * RPT

    def zc(k, carry):
        pltpu.sync_copy(zbuf, acc.at[pl.ds(tbase + k * 32, 32)])
        return carry

    lax.fori_loop(0, RPT // 32, zc, 0)
    pltpu.sync_copy(zbuf.at[pl.ds(0, RPT % 32)],
                    acc.at[pl.ds(tbase + (RPT // 32) * 32, RPT % 32)])
    plsc.subcore_barrier()

    def loads(i, sl):
        row = wid * NCH + i
        pltpu.async_copy(p1.at[pl.ds(row * C, C)], pb.at[sl], semL[sl])
        pltpu.async_copy(p2.at[pl.ds(row * C, C)], qb.at[sl], semL[sl])
        pltpu.async_copy(idxd.at[row], iv.at[sl], semL[sl])
        pltpu.async_copy(idx2.at[row], iv2.at[sl], semL[sl])

    def drainL(i, sl):
        row = wid * NCH + i
        pltpu.make_async_copy(p1.at[pl.ds(row * C, C)], pb.at[sl], semL[sl]).wait()
        pltpu.make_async_copy(p2.at[pl.ds(row * C, C)], qb.at[sl], semL[sl]).wait()
        pltpu.make_async_copy(idxd.at[row], iv.at[sl], semL[sl]).wait()
        pltpu.make_async_copy(idx2.at[row], iv2.at[sl], semL[sl]).wait()

    def fire(sl):
        pltpu.async_copy(pb.at[sl], acc.at[iv.at[sl]], semA[sl], add=True)
        pltpu.async_copy(qb.at[sl], acc.at[iv2.at[sl]], semA[sl], add=True)

    def drainA(sl):
        pltpu.make_async_copy(pb.at[sl], acc.at[iv.at[sl]], semA[sl]).wait()
        pltpu.make_async_copy(qb.at[sl], acc.at[iv2.at[sl]], semA[sl]).wait()

    loads(0, 0)
    loads(1, 1)

    def body2(k, carry):
        i0 = 2 * k
        drainL(i0, 0)
        fire(0)
        drainL(i0 + 1, 1)
        fire(1)
        drainA(0)

        @pl.when(i0 + 2 < NCH)
        def _():
            loads(i0 + 2, 0)

        drainA(1)

        @pl.when(i0 + 3 < NCH)
        def _():
            loads(i0 + 3, 1)

        return carry

    lax.fori_loop(0, NCH // 2, body2, 0)
    drainL(NCH - 1, 0)
    fire(0)
    drainA(0)
    plsc.subcore_barrier()
    pltpu.sync_copy(acc.at[pl.ds(tbase, RPT)], out.at[pl.ds(c * NPP + tbase, RPT)])


_sc_scatter = pl.kernel(
    _sc_scatter_body,
    out_type=jax.ShapeDtypeStruct((NC * NPP, D), _f32),
    mesh=plsc.VectorSubcoreMesh(
        core_axis_name="c", subcore_axis_name="s", num_cores=NC, num_subcores=NS
    ),
    scratch_types=[
        pltpu.VMEM_SHARED((NPP, D), _f32),
        pltpu.VMEM((2, C, D), _f32),
        pltpu.VMEM((2, C, D), _f32),
        pltpu.VMEM((2, C), jnp.int32),
        pltpu.VMEM((2, C), jnp.int32),
        pltpu.VMEM((32, D), _f32),
        pltpu.SemaphoreType.DMA,
        pltpu.SemaphoreType.DMA,
        pltpu.SemaphoreType.DMA,
        pltpu.SemaphoreType.DMA,
    ],
)


def _fe_body(we, we1e, be, be1, fe, fb):
    for l in range(3):
        w1e = we1e[l]
        fe[l] = jnp.dot(we[...], w1e, preferred_element_type=_f32)
        fb[l] = jnp.dot(be[...], w1e, preferred_element_type=_f32) + be1[l][None, :]


_fe_call = pl.pallas_call(
    _fe_body,
    out_shape=(
        jax.ShapeDtypeStruct((3, NRBF, D), _f32),
        jax.ShapeDtypeStruct((3, 1, D), _f32),
    ),
)


def _init_body(nf_ref, pos_ref, wn, bn, wda, wsb, h_ref, td_ref, ts_ref):
    nf = nf_ref[...]
    nf = jnp.concatenate([nf[:, :6], nf[:, 6:7] * (1.0 / RES_SCALE)], axis=1)
    h = jnp.dot(nf, wn[...], preferred_element_type=_f32) + bn[...]
    h_ref[...] = h
    a = jnp.dot(h, wda[...], preferred_element_type=_f32)
    b = jnp.dot(h, wsb[...], preferred_element_type=_f32)
    p = pos_ref[...]
    z = jnp.zeros((BN, WG - D - 3), _f32)
    td_ref[...] = jnp.concatenate([a, p, z], axis=1)
    ts_ref[...] = jnp.concatenate([b, -p, z], axis=1)


_init_call = pl.pallas_call(
    _init_body,
    grid=(N // BN,),
    in_specs=[
        pl.BlockSpec((BN, 7), lambda i: (i, 0)),
        pl.BlockSpec((BN, 3), lambda i: (i, 0)),
        pl.BlockSpec((7, D), lambda i: (0, 0)),
        pl.BlockSpec((1, D), lambda i: (0, 0)),
        pl.BlockSpec((D, D), lambda i: (0, 0)),
        pl.BlockSpec((D, D), lambda i: (0, 0)),
    ],
    out_specs=[
        pl.BlockSpec((BN, D), lambda i: (i, 0)),
        pl.BlockSpec((BN, WG), lambda i: (i, 0)),
        pl.BlockSpec((BN, WG), lambda i: (i, 0)),
    ],
    out_shape=[
        jax.ShapeDtypeStruct((N, D), _f32),
        jax.ShapeDtypeStruct((N, WG), _f32),
        jax.ShapeDtypeStruct((N, WG), _f32),
    ],
)


def _edge_body(g_ref, ea_ref, dst_ref, fe, fb, wd2, we2, be2, wx1, bx1, wx2t,
               bx2, p1_ref, p2_ref):
    x = g_ref[...]
    gsum = x[:, :D]
    rel = x[:, D:D + 3]
    d2 = jnp.sum(rel * rel, axis=1, keepdims=True)
    dd = ea_ref[...]
    cen = (lax.broadcasted_iota(jnp.int32, (1, NRBF), 1).astype(_f32)
           * (RMAX / (NRBF - 1)))
    rbf = jnp.exp(-GAMMA * (dd - cen) ** 2)
    pre = (gsum + d2 * wd2[...]
           + jnp.dot(rbf, fe[...], preferred_element_type=_f32) + fb[...])
    m = _silu(pre)
    m = _silu(jnp.dot(m, we2[...], preferred_element_type=_f32) + be2[...])
    t = _silu(jnp.dot(m, wx1[...], preferred_element_type=_f32) + bx1[...])
    w = jnp.sum(t * wx2t[...], axis=1, keepdims=True) + bx2[...]
    p1_ref[...] = m
    rw = rel * w
    dm = lax.rem(dst_ref[...], jnp.full((BE, 1), 32, jnp.int32))
    lane = lax.broadcasted_iota(jnp.int32, (1, D), 1)
    lm = lax.rem(lane, jnp.full((1, D), 4, jnp.int32))
    grp = lax.div(lane, jnp.full((1, D), 4, jnp.int32))
    vals = (rw[:, 0:1] * (lm == 0).astype(_f32)
            + rw[:, 1:2] * (lm == 1).astype(_f32)
            + rw[:, 2:3] * (lm == 2).astype(_f32)
            + (lm == 3).astype(_f32))
    p2_ref[...] = jnp.where(grp == dm, vals, 0.0)


_edge_call = pl.pallas_call(
    _edge_body,
    grid=(E // BE,),
    in_specs=[
        pl.BlockSpec((BE, WO), lambda i: (i, 0)),
        pl.BlockSpec((BE, 1), lambda i: (i, 0)),
        pl.BlockSpec((BE, 1), lambda i: (i, 0)),
        pl.BlockSpec((NRBF, D), lambda i: (0, 0)),
        pl.BlockSpec((1, D), lambda i: (0, 0)),
        pl.BlockSpec((1, D), lambda i: (0, 0)),
        pl.BlockSpec((D, D), lambda i: (0, 0)),
        pl.BlockSpec((1, D), lambda i: (0, 0)),
        pl.BlockSpec((D, D), lambda i: (0, 0)),
        pl.BlockSpec((1, D), lambda i: (0, 0)),
        pl.BlockSpec((1, D), lambda i: (0, 0)),
        pl.BlockSpec((1, 1), lambda i: (0, 0)),
    ],
    out_specs=[
        pl.BlockSpec((BE, D), lambda i: (i, 0)),
        pl.BlockSpec((BE, D), lambda i: (i, 0)),
    ],
    out_shape=[
        jax.ShapeDtypeStruct((E, D), _f32),
        jax.ShapeDtypeStruct((E, D), _f32),
    ],
)


def _node_body(a0_ref, a1_ref, pd0_ref, pd1_ref, h_ref, pos_ref, wh1a, wh1b,
               bh1, wh2, bh2, lg, lb, *rest, with_tables):
    if with_tables:
        wda, wsb, hn_ref, pn_ref, td_ref, ts_ref = rest
    else:
        hn_ref, pn_ref = rest
    agg = a0_ref[...] + a1_ref[...]
    pacc = pd0_ref[...] + pd1_ref[...]
    posd = pacc[:, :3]
    deg = pacc[:, 3:4]
    pn = pos_ref[...] + posd / (deg + 1.0)
    hh = h_ref[...]
    u = _silu(jnp.dot(hh, wh1a[...], preferred_element_type=_f32)
              + jnp.dot(agg, wh1b[...], preferred_element_type=_f32) + bh1[...])
    hn = hh + jnp.dot(u, wh2[...], preferred_element_type=_f32) + bh2[...]
    mu = jnp.mean(hn, axis=1, keepdims=True)
    var = jnp.mean((hn - mu) ** 2, axis=1, keepdims=True)
    hn = (hn - mu) * lax.rsqrt(var + 1e-5) * lg[...] + lb[...]
    hn_ref[...] = hn
    pn_ref[...] = pn
    if with_tables:
        a = jnp.dot(hn, wda[...], preferred_element_type=_f32)
        b = jnp.dot(hn, wsb[...], preferred_element_type=_f32)
        z = jnp.zeros((BN, WG - D - 3), _f32)
        td_ref[...] = jnp.concatenate([a, pn, z], axis=1)
        ts_ref[...] = jnp.concatenate([b, -pn, z], axis=1)


def _make_node_call(with_tables):
    n_extra_in = 2 if with_tables else 0
    out_shapes = [
        jax.ShapeDtypeStruct((N, D), _f32),
        jax.ShapeDtypeStruct((N, 3), _f32),
    ]
    out_specs = [
        pl.BlockSpec((BN, D), lambda i: (i, 0)),
        pl.BlockSpec((BN, 3), lambda i: (i, 0)),
    ]
    if with_tables:
        out_shapes += [jax.ShapeDtypeStruct((N, WG), _f32)] * 2
        out_specs += [pl.BlockSpec((BN, WG), lambda i: (i, 0))] * 2
    return pl.pallas_call(
        functools.partial(_node_body, with_tables=with_tables),
        grid=(N // BN,),
        in_specs=[
            pl.BlockSpec((BN, D), lambda i: (i, 0)),
            pl.BlockSpec((BN, D), lambda i: (i, 0)),
            pl.BlockSpec((BN, 4), lambda i: (i, 0)),
            pl.BlockSpec((BN, 4), lambda i: (i, 0)),
            pl.BlockSpec((BN, D), lambda i: (i, 0)),
            pl.BlockSpec((BN, 3), lambda i: (i, 0)),
            pl.BlockSpec((D, D), lambda i: (0, 0)),
            pl.BlockSpec((D, D), lambda i: (0, 0)),
            pl.BlockSpec((1, D), lambda i: (0, 0)),
            pl.BlockSpec((D, D), lambda i: (0, 0)),
            pl.BlockSpec((1, D), lambda i: (0, 0)),
            pl.BlockSpec((1, D), lambda i: (0, 0)),
            pl.BlockSpec((1, D), lambda i: (0, 0)),
        ] + [pl.BlockSpec((D, D), lambda i: (0, 0))] * n_extra_in,
        out_specs=out_specs,
        out_shape=out_shapes,
    )


_node_mid = _make_node_call(True)
_node_last = _make_node_call(False)


def kernel(node_feat, edge_attr, pos, Wn, bn, We, be, We1, be1, We2, be2,
           Wx1, bx1, Wx2, bx2, Wh1, bh1, Wh2, bh2, ln_g, ln_b, edge_index):
    src = edge_index[0]
    dst = edge_index[1]
    idxd = dst
    idxs = src
    idxd2 = dst.reshape(NW * NCH, C)
    idx2 = (PB + dst // 32).reshape(NW * NCH, C)
    dstc = dst.reshape(E, 1)

    fe, fb = _fe_call(We, We1[:, 2 * D + 1:, :], be.reshape(1, D), be1)
    h, td, ts = _init_call(node_feat, pos, Wn, bn.reshape(1, D),
                           We1[0, :D, :], We1[0, D:2 * D, :])
    for l in range(3):
        g = _sc_gather(td, ts, idxd, idxs)
        p1, p2 = _edge_call(g, edge_attr, dstc, fe[l], fb[l],
                            We1[l, 2 * D, :].reshape(1, D), We2[l],
                            be2[l].reshape(1, D), Wx1[l], bx1[l].reshape(1, D),
                            Wx2[l].reshape(1, D), bx2[l].reshape(1, 1))
        outm = _sc_scatter(p1, p2, idxd2, idx2)
        a0 = outm[:N]
        a1 = outm[NPP:NPP + N]
        pd0 = outm[PB:PB + NPOS].reshape(NPOS * 32, 4)[:N]
        pd1 = outm[NPP + PB:NPP + PB + NPOS].reshape(NPOS * 32, 4)[:N]
        if l < 2:
            h, pos, td, ts = _node_mid(
                a0, a1, pd0, pd1, h, pos,
                Wh1[l, :D, :], Wh1[l, D:, :], bh1[l].reshape(1, D),
                Wh2[l], bh2[l].reshape(1, D),
                ln_g[l].reshape(1, D), ln_b[l].reshape(1, D),
                We1[l + 1, :D, :], We1[l + 1, D:2 * D, :])
        else:
            h, pos = _node_last(
                a0, a1, pd0, pd1, h, pos,
                Wh1[l, :D, :], Wh1[l, D:, :], bh1[l].reshape(1, D),
                Wh2[l], bh2[l].reshape(1, D),
                ln_g[l].reshape(1, D), ln_b[l].reshape(1, D))
    return h, pos


# speedup vs baseline: 3.2373x; 1.1980x over previous
import functools

import jax
import jax.numpy as jnp
from jax import lax
from jax.experimental import pallas as pl
from jax.experimental.pallas import tpu as pltpu
from jax.experimental.pallas import tpu_sc as plsc

N = 10000
E = 320000
D = 128
WG = 256
WO = 144
NRBF = 32
RMAX = 10.0
GAMMA = 1.0 / ((RMAX / NRBF) ** 2)
RES_SCALE = 1000.0

NC = 2
NS = 16
NW = NC * NS
EPW = E // NW
C = 80
NCH = EPW // C
NHLF = 2
E2 = E // NHLF
EPW2 = E2 // NW
C2 = 40
NCH2 = EPW2 // C2
PB = 10000
NPOS = 320
NPP = 10368
RPT = NPP // NS

BE = 512
BN = 1000

_f32 = jnp.float32


def _silu(x):
    return x * jax.nn.sigmoid(x)


def _make_sc_gather(ne, c_):
    epw = ne // NW
    nch = epw // c_

    def body(td, ts, idxd, idxs, g, idb, isb, bufd, bufs, obuf,
             semd0, sems0, semd1, sems1, semw0):
        c = lax.axis_index("c")
        s = lax.axis_index("s")
        wid = c * NS + s
        sems = ((semd0, sems0), (semd1, sems1))

        pltpu.sync_copy(idxd.at[pl.ds(wid * epw, epw)], idb)
        pltpu.sync_copy(idxs.at[pl.ds(wid * epw, epw)], isb)

        def start(i, sl):
            pltpu.async_copy(td.at[idb.at[pl.ds(i * c_, c_)]], bufd.at[sl],
                             sems[sl][0])
            pltpu.async_copy(ts.at[isb.at[pl.ds(i * c_, c_)]], bufs.at[sl],
                             sems[sl][1])

        def finish(i, sl):
            row = wid * nch + i
            pltpu.make_async_copy(td.at[idb.at[pl.ds(i * c_, c_)]], bufd.at[sl],
                                  sems[sl][0]).wait()
            pltpu.make_async_copy(ts.at[isb.at[pl.ds(i * c_, c_)]], bufs.at[sl],
                                  sems[sl][1]).wait()

            @pl.when(i >= 1)
            def _():
                pltpu.make_async_copy(obuf, g.at[pl.ds(row * c_, c_)],
                                      semw0).wait()

            def add_row(r, carry):
                for j in range(WO // 16):
                    obuf[r, pl.ds(j * 16, 16)] = (
                        bufd[sl, r, pl.ds(j * 16, 16)]
                        + bufs[sl, r, pl.ds(j * 16, 16)])
                return carry

            lax.fori_loop(0, c_, add_row, 0)
            pltpu.async_copy(obuf, g.at[pl.ds(row * c_, c_)], semw0)

        start(0, 0)

        def body2(k, carry):
            i0 = 2 * k
            start(i0 + 1, 1)
            finish(i0, 0)
            start(i0 + 2, 0)
            finish(i0 + 1, 1)
            return carry

        lax.fori_loop(0, (nch - 1) // 2, body2, 0)
        finish(nch - 1, 0)
        pltpu.make_async_copy(obuf, g.at[pl.ds(0, c_)], semw0).wait()

    return pl.kernel(
        body,
        out_type=jax.ShapeDtypeStruct((ne, WO), _f32),
        mesh=plsc.VectorSubcoreMesh(
            core_axis_name="c", subcore_axis_name="s",
            num_cores=NC, num_subcores=NS,
        ),
        scratch_types=[
            pltpu.VMEM((epw,), jnp.int32),
            pltpu.VMEM((epw,), jnp.int32),
            pltpu.VMEM((2, c_, WG), _f32),
            pltpu.VMEM((2, c_, WG), _f32),
            pltpu.VMEM((c_, WO), _f32),
            pltpu.SemaphoreType.DMA,
            pltpu.SemaphoreType.DMA,
            pltpu.SemaphoreType.DMA,
            pltpu.SemaphoreType.DMA,
            pltpu.SemaphoreType.DMA,
        ],
    )


_sc_gather_h = _make_sc_gather(E2, C2)


def _make_sc_scatter(ne, c_):
    epw = ne // NW
    nch = epw // c_

    def body(p1, p2, idxd, idx2, out, acc, pb, qb, iv, iv2, zbuf,
             semA0, semA1, semL0, semL1):
        c = lax.axis_index("c")
        s = lax.axis_index("s")
        wid = c * NS + s
        semA = (semA0, semA1)
        semL = (semL0, semL1)

        def zrow(r, carry):
            for j in range(D // 16):
                zbuf[r, pl.ds(j * 16, 16)] = jnp.zeros((16,), _f32)
            return carry

        lax.fori_loop(0, 32, zrow, 0)
        tbase = s * RPT

        def zc(k, carry):
            pltpu.sync_copy(zbuf, acc.at[pl.ds(tbase + k * 32, 32)])
            return carry

        lax.fori_loop(0, RPT // 32, zc, 0)
        pltpu.sync_copy(zbuf.at[pl.ds(0, RPT % 32)],
                        acc.at[pl.ds(tbase + (RPT // 32) * 32, RPT % 32)])
        plsc.subcore_barrier()

        def loads(i, sl):
            row = wid * nch + i
            pltpu.async_copy(p1.at[pl.ds(row * c_, c_)], pb.at[sl], semL[sl])
            pltpu.async_copy(p2.at[pl.ds(row * c_, c_)], qb.at[sl], semL[sl])
            pltpu.async_copy(idxd.at[row], iv.at[sl], semL[sl])
            pltpu.async_copy(idx2.at[row], iv2.at[sl], semL[sl])

        def drainL(i, sl):
            row = wid * nch + i
            pltpu.make_async_copy(p1.at[pl.ds(row * c_, c_)], pb.at[sl],
                                  semL[sl]).wait()
            pltpu.make_async_copy(p2.at[pl.ds(row * c_, c_)], qb.at[sl],
                                  semL[sl]).wait()
            pltpu.make_async_copy(idxd.at[row], iv.at[sl], semL[sl]).wait()
            pltpu.make_async_copy(idx2.at[row], iv2.at[sl], semL[sl]).wait()

        def fire(sl):
            pltpu.async_copy(pb.at[sl], acc.at[iv.at[sl]], semA[sl], add=True)
            pltpu.async_copy(qb.at[sl], acc.at[iv2.at[sl]], semA[sl], add=True)

        def drainA(sl):
            pltpu.make_async_copy(pb.at[sl], acc.at[iv.at[sl]], semA[sl]).wait()
            pltpu.make_async_copy(qb.at[sl], acc.at[iv2.at[sl]], semA[sl]).wait()

        loads(0, 0)
        loads(1, 1)

        def body2(k, carry):
            i0 = 2 * k
            drainL(i0, 0)
            fire(0)
            drainL(i0 + 1, 1)
            fire(1)
            drainA(0)

            @pl.when(i0 + 2 < nch)
            def _():
                loads(i0 + 2, 0)

            drainA(1)

            @pl.when(i0 + 3 < nch)
            def _():
                loads(i0 + 3, 1)

            return carry

        lax.fori_loop(0, nch // 2, body2, 0)
        drainL(nch - 1, 0)
        fire(0)
        drainA(0)
        plsc.subcore_barrier()
        pltpu.sync_copy(acc.at[pl.ds(tbase, RPT)],
                        out.at[pl.ds(c * NPP + tbase, RPT)])

    return pl.kernel(
        body,
        out_type=jax.ShapeDtypeStruct((NC * NPP, D), _f32),
        mesh=plsc.VectorSubcoreMesh(
            core_axis_name="c", subcore_axis_name="s",
            num_cores=NC, num_subcores=NS,
        ),
        scratch_types=[
            pltpu.VMEM_SHARED((NPP, D), _f32),
            pltpu.VMEM((2, c_, D), _f32),
            pltpu.VMEM((2, c_, D), _f32),
            pltpu.VMEM((2, c_), jnp.int32),
            pltpu.VMEM((2, c_), jnp.int32),
            pltpu.VMEM((32, D), _f32),
            pltpu.SemaphoreType.DMA,
            pltpu.SemaphoreType.DMA,
            pltpu.SemaphoreType.DMA,
            pltpu.SemaphoreType.DMA,
        ],
    )


_sc_scatter_h = _make_sc_scatter(E2, C2)


def _fe_body(we, we1e, be, be1, fe, fb):
    for l in range(3):
        w1e = we1e[l]
        fe[l] = jnp.dot(we[...], w1e, preferred_element_type=_f32)
        fb[l] = jnp.dot(be[...], w1e, preferred_element_type=_f32) + be1[l][None, :]


_fe_call = pl.pallas_call(
    _fe_body,
    out_shape=(
        jax.ShapeDtypeStruct((3, NRBF, D), _f32),
        jax.ShapeDtypeStruct((3, 1, D), _f32),
    ),
)


def _init_body(nf_ref, pos_ref, wn, bn, wda, wsb, h_ref, td_ref, ts_ref):
    nf = nf_ref[...]
    nf = jnp.concatenate([nf[:, :6], nf[:, 6:7] * (1.0 / RES_SCALE)], axis=1)
    h = jnp.dot(nf, wn[...], preferred_element_type=_f32) + bn[...]
    h_ref[...] = h
    a = jnp.dot(h, wda[...], preferred_element_type=_f32)
    b = jnp.dot(h, wsb[...], preferred_element_type=_f32)
    p = pos_ref[...]
    z = jnp.zeros((BN, WG - D - 3), _f32)
    td_ref[...] = jnp.concatenate([a, p, z], axis=1)
    ts_ref[...] = jnp.concatenate([b, -p, z], axis=1)


_init_call = pl.pallas_call(
    _init_body,
    grid=(N // BN,),
    in_specs=[
        pl.BlockSpec((BN, 7), lambda i: (i, 0)),
        pl.BlockSpec((BN, 3), lambda i: (i, 0)),
        pl.BlockSpec((7, D), lambda i: (0, 0)),
        pl.BlockSpec((1, D), lambda i: (0, 0)),
        pl.BlockSpec((D, D), lambda i: (0, 0)),
        pl.BlockSpec((D, D), lambda i: (0, 0)),
    ],
    out_specs=[
        pl.BlockSpec((BN, D), lambda i: (i, 0)),
        pl.BlockSpec((BN, WG), lambda i: (i, 0)),
        pl.BlockSpec((BN, WG), lambda i: (i, 0)),
    ],
    out_shape=[
        jax.ShapeDtypeStruct((N, D), _f32),
        jax.ShapeDtypeStruct((N, WG), _f32),
        jax.ShapeDtypeStruct((N, WG), _f32),
    ],
)


def _edge_body(g_ref, ea_ref, dst_ref, fe, fb, wd2, we2, be2, wx1, bx1, wx2t,
               bx2, p1_ref, p2_ref):
    x = g_ref[...]
    gsum = x[:, :D]
    rel = x[:, D:D + 3]
    d2 = jnp.sum(rel * rel, axis=1, keepdims=True)
    dd = ea_ref[...]
    cen = (lax.broadcasted_iota(jnp.int32, (1, NRBF), 1).astype(_f32)
           * (RMAX / (NRBF - 1)))
    rbf = jnp.exp(-GAMMA * (dd - cen) ** 2)
    pre = (gsum + d2 * wd2[...]
           + jnp.dot(rbf, fe[...], preferred_element_type=_f32) + fb[...])
    m = _silu(pre)
    m = _silu(jnp.dot(m, we2[...], preferred_element_type=_f32) + be2[...])
    t = _silu(jnp.dot(m, wx1[...], preferred_element_type=_f32) + bx1[...])
    w = jnp.sum(t * wx2t[...], axis=1, keepdims=True) + bx2[...]
    p1_ref[...] = m
    rw = rel * w
    be = dst_ref.shape[0]
    dm = lax.rem(dst_ref[...], jnp.full((be, 1), 32, jnp.int32))
    lane = lax.broadcasted_iota(jnp.int32, (1, D), 1)
    lm = lax.rem(lane, jnp.full((1, D), 4, jnp.int32))
    grp = lax.div(lane, jnp.full((1, D), 4, jnp.int32))
    vals = (rw[:, 0:1] * (lm == 0).astype(_f32)
            + rw[:, 1:2] * (lm == 1).astype(_f32)
            + rw[:, 2:3] * (lm == 2).astype(_f32)
            + (lm == 3).astype(_f32))
    p2_ref[...] = jnp.where(grp == dm, vals, 0.0)


def _make_edge_call(ne, be):
    return pl.pallas_call(
        _edge_body,
        grid=(ne // be,),
        in_specs=[
            pl.BlockSpec((be, WO), lambda i: (i, 0)),
            pl.BlockSpec((be, 1), lambda i: (i, 0)),
            pl.BlockSpec((be, 1), lambda i: (i, 0)),
            pl.BlockSpec((NRBF, D), lambda i: (0, 0)),
            pl.BlockSpec((1, D), lambda i: (0, 0)),
            pl.BlockSpec((1, D), lambda i: (0, 0)),
            pl.BlockSpec((D, D), lambda i: (0, 0)),
            pl.BlockSpec((1, D), lambda i: (0, 0)),
            pl.BlockSpec((D, D), lambda i: (0, 0)),
            pl.BlockSpec((1, D), lambda i: (0, 0)),
            pl.BlockSpec((1, D), lambda i: (0, 0)),
            pl.BlockSpec((1, 1), lambda i: (0, 0)),
        ],
        out_specs=[
            pl.BlockSpec((be, D), lambda i: (i, 0)),
            pl.BlockSpec((be, D), lambda i: (i, 0)),
        ],
        out_shape=[
            jax.ShapeDtypeStruct((ne, D), _f32),
            jax.ShapeDtypeStruct((ne, D), _f32),
        ],
    )


_edge_call_h = _make_edge_call(E2, 640)


def _node_body(a0_ref, a1_ref, a2_ref, a3_ref, pd0_ref, pd1_ref, pd2_ref,
               pd3_ref, h_ref, pos_ref, wh1a, wh1b,
               bh1, wh2, bh2, lg, lb, *rest, with_tables):
    if with_tables:
        wda, wsb, hn_ref, pn_ref, td_ref, ts_ref = rest
    else:
        hn_ref, pn_ref = rest
    agg = (a0_ref[...] + a1_ref[...]) + (a2_ref[...] + a3_ref[...])
    pacc = (pd0_ref[...] + pd1_ref[...]) + (pd2_ref[...] + pd3_ref[...])
    posd = pacc[:, :3]
    deg = pacc[:, 3:4]
    pn = pos_ref[...] + posd / (deg + 1.0)
    hh = h_ref[...]
    u = _silu(jnp.dot(hh, wh1a[...], preferred_element_type=_f32)
              + jnp.dot(agg, wh1b[...], preferred_element_type=_f32) + bh1[...])
    hn = hh + jnp.dot(u, wh2[...], preferred_element_type=_f32) + bh2[...]
    mu = jnp.mean(hn, axis=1, keepdims=True)
    var = jnp.mean((hn - mu) ** 2, axis=1, keepdims=True)
    hn = (hn - mu) * lax.rsqrt(var + 1e-5) * lg[...] + lb[...]
    hn_ref[...] = hn
    pn_ref[...] = pn
    if with_tables:
        a = jnp.dot(hn, wda[...], preferred_element_type=_f32)
        b = jnp.dot(hn, wsb[...], preferred_element_type=_f32)
        z = jnp.zeros((BN, WG - D - 3), _f32)
        td_ref[...] = jnp.concatenate([a, pn, z], axis=1)
        ts_ref[...] = jnp.concatenate([b, -pn, z], axis=1)


def _make_node_call(with_tables):
    n_extra_in = 2 if with_tables else 0
    out_shapes = [
        jax.ShapeDtypeStruct((N, D), _f32),
        jax.ShapeDtypeStruct((N, 3), _f32),
    ]
    out_specs = [
        pl.BlockSpec((BN, D), lambda i: (i, 0)),
        pl.BlockSpec((BN, 3), lambda i: (i, 0)),
    ]
    if with_tables:
        out_shapes += [jax.ShapeDtypeStruct((N, WG), _f32)] * 2
        out_specs += [pl.BlockSpec((BN, WG), lambda i: (i, 0))] * 2
    return pl.pallas_call(
        functools.partial(_node_body, with_tables=with_tables),
        grid=(N // BN,),
        in_specs=[
            pl.BlockSpec((BN, D), lambda i: (i, 0)),
            pl.BlockSpec((BN, D), lambda i: (i, 0)),
            pl.BlockSpec((BN, D), lambda i: (i, 0)),
            pl.BlockSpec((BN, D), lambda i: (i, 0)),
            pl.BlockSpec((BN, 4), lambda i: (i, 0)),
            pl.BlockSpec((BN, 4), lambda i: (i, 0)),
            pl.BlockSpec((BN, 4), lambda i: (i, 0)),
            pl.BlockSpec((BN, 4), lambda i: (i, 0)),
            pl.BlockSpec((BN, D), lambda i: (i, 0)),
            pl.BlockSpec((BN, 3), lambda i: (i, 0)),
            pl.BlockSpec((D, D), lambda i: (0, 0)),
            pl.BlockSpec((D, D), lambda i: (0, 0)),
            pl.BlockSpec((1, D), lambda i: (0, 0)),
            pl.BlockSpec((D, D), lambda i: (0, 0)),
            pl.BlockSpec((1, D), lambda i: (0, 0)),
            pl.BlockSpec((1, D), lambda i: (0, 0)),
            pl.BlockSpec((1, D), lambda i: (0, 0)),
        ] + [pl.BlockSpec((D, D), lambda i: (0, 0))] * n_extra_in,
        out_specs=out_specs,
        out_shape=out_shapes,
    )


_node_mid = _make_node_call(True)
_node_last = _make_node_call(False)


def kernel(node_feat, edge_attr, pos, Wn, bn, We, be, We1, be1, We2, be2,
           Wx1, bx1, Wx2, bx2, Wh1, bh1, Wh2, bh2, ln_g, ln_b, edge_index):
    src = edge_index[0]
    dst = edge_index[1]
    halves = []
    for hf in range(NHLF):
        lo = hf * E2
        d_h = lax.slice_in_dim(dst, lo, lo + E2)
        s_h = lax.slice_in_dim(src, lo, lo + E2)
        halves.append(dict(
            dst=d_h, src=s_h,
            idxd2=d_h.reshape(NW * NCH2, C2),
            idx2=(PB + d_h // 32).reshape(NW * NCH2, C2),
            ea=lax.slice_in_dim(edge_attr, lo, lo + E2),
            dstc=d_h.reshape(E2, 1),
        ))

    fe, fb = _fe_call(We, We1[:, 2 * D + 1:, :], be.reshape(1, D), be1)
    h, td, ts = _init_call(node_feat, pos, Wn, bn.reshape(1, D),
                           We1[0, :D, :], We1[0, D:2 * D, :])
    for l in range(3):
        gs = [_sc_gather_h(td, ts, hv["dst"], hv["src"]) for hv in halves]
        ps = [_edge_call_h(g, hv["ea"], hv["dstc"], fe[l], fb[l],
                           We1[l, 2 * D, :].reshape(1, D), We2[l],
                           be2[l].reshape(1, D), Wx1[l], bx1[l].reshape(1, D),
                           Wx2[l].reshape(1, D), bx2[l].reshape(1, 1))
              for g, hv in zip(gs, halves)]
        outs = [_sc_scatter_h(p1, p2, hv["idxd2"], hv["idx2"])
                for (p1, p2), hv in zip(ps, halves)]
        accs = []
        pds = []
        for outm in outs:
            accs += [outm[:N], outm[NPP:NPP + N]]
            pds += [outm[PB:PB + NPOS].reshape(NPOS * 32, 4)[:N],
                    outm[NPP + PB:NPP + PB + NPOS].reshape(NPOS * 32, 4)[:N]]
        if l < 2:
            h, pos, td, ts = _node_mid(
                *accs, *pds, h, pos,
                Wh1[l, :D, :], Wh1[l, D:, :], bh1[l].reshape(1, D),
                Wh2[l], bh2[l].reshape(1, D),
                ln_g[l].reshape(1, D), ln_b[l].reshape(1, D),
                We1[l + 1, :D, :], We1[l + 1, D:2 * D, :])
        else:
            h, pos = _node_last(
                *accs, *pds, h, pos,
                Wh1[l, :D, :], Wh1[l, D:, :], bh1[l].reshape(1, D),
                Wh2[l], bh2[l].reshape(1, D),
                ln_g[l].reshape(1, D), ln_b[l].reshape(1, D))
    return h, pos


# speedup vs baseline: 3.2637x; 1.0082x over previous
import functools

import jax
import jax.numpy as jnp
from jax import lax
from jax.experimental import pallas as pl
from jax.experimental.pallas import tpu as pltpu
from jax.experimental.pallas import tpu_sc as plsc

N = 10000
E = 320000
D = 128
WG = 256
WO = 144
NRBF = 32
RMAX = 10.0
GAMMA = 1.0 / ((RMAX / NRBF) ** 2)
RES_SCALE = 1000.0

NC = 2
NS = 16
NW = NC * NS
EPW = E // NW
C = 80
NCH = EPW // C
NHLF = 2
E2 = E // NHLF
EPW2 = E2 // NW
C2 = 40
NCH2 = EPW2 // C2
PB = 10000
NPOS = 320
NPP = 10368
RPT = NPP // NS

BE = 512
BN = 1000

_f32 = jnp.float32


def _silu(x):
    return x * jax.nn.sigmoid(x)


def _make_sc_gather(ne, c_):
    epw = ne // NW
    nch = epw // c_

    def body(td, ts, idxd, idxs, g, idb, isb, bufd, bufs, obuf,
             semd0, sems0, semd1, sems1, semw0):
        c = lax.axis_index("c")
        s = lax.axis_index("s")
        wid = c * NS + s
        sems = ((semd0, sems0), (semd1, sems1))

        pltpu.sync_copy(idxd.at[pl.ds(wid * epw, epw)], idb)
        pltpu.sync_copy(idxs.at[pl.ds(wid * epw, epw)], isb)

        def start(i, sl):
            pltpu.async_copy(td.at[idb.at[pl.ds(i * c_, c_)]], bufd.at[sl],
                             sems[sl][0])
            pltpu.async_copy(ts.at[isb.at[pl.ds(i * c_, c_)]], bufs.at[sl],
                             sems[sl][1])

        def finish(i, sl):
            row = wid * nch + i
            pltpu.make_async_copy(td.at[idb.at[pl.ds(i * c_, c_)]], bufd.at[sl],
                                  sems[sl][0]).wait()
            pltpu.make_async_copy(ts.at[isb.at[pl.ds(i * c_, c_)]], bufs.at[sl],
                                  sems[sl][1]).wait()

            @pl.when(i >= 1)
            def _():
                pltpu.make_async_copy(obuf, g.at[pl.ds(row * c_, c_)],
                                      semw0).wait()

            def add_row(r, carry):
                for j in range(WO // 16):
                    obuf[r, pl.ds(j * 16, 16)] = (
                        bufd[sl, r, pl.ds(j * 16, 16)]
                        + bufs[sl, r, pl.ds(j * 16, 16)])
                return carry

            lax.fori_loop(0, c_, add_row, 0)
            pltpu.async_copy(obuf, g.at[pl.ds(row * c_, c_)], semw0)

        start(0, 0)

        def body2(k, carry):
            i0 = 2 * k
            start(i0 + 1, 1)
            finish(i0, 0)
            start(i0 + 2, 0)
            finish(i0 + 1, 1)
            return carry

        lax.fori_loop(0, (nch - 1) // 2, body2, 0)
        finish(nch - 1, 0)
        pltpu.make_async_copy(obuf, g.at[pl.ds(0, c_)], semw0).wait()

    return pl.kernel(
        body,
        out_type=jax.ShapeDtypeStruct((ne, WO), _f32),
        mesh=plsc.VectorSubcoreMesh(
            core_axis_name="c", subcore_axis_name="s",
            num_cores=NC, num_subcores=NS,
        ),
        scratch_types=[
            pltpu.VMEM((epw,), jnp.int32),
            pltpu.VMEM((epw,), jnp.int32),
            pltpu.VMEM((2, c_, WG), _f32),
            pltpu.VMEM((2, c_, WG), _f32),
            pltpu.VMEM((c_, WO), _f32),
            pltpu.SemaphoreType.DMA,
            pltpu.SemaphoreType.DMA,
            pltpu.SemaphoreType.DMA,
            pltpu.SemaphoreType.DMA,
            pltpu.SemaphoreType.DMA,
        ],
    )


_sc_gather_h = _make_sc_gather(E2, C2)


def _make_sc_scatter(ne, c_):
    epw = ne // NW
    nch = epw // c_

    def body(p1, p2, idxd, idx2, out, acc, pb, qb, iv, iv2, zbuf,
             semA0, semA1, semL0, semL1):
        c = lax.axis_index("c")
        s = lax.axis_index("s")
        wid = c * NS + s
        semA = (semA0, semA1)
        semL = (semL0, semL1)

        def zrow(r, carry):
            for j in range(D // 16):
                zbuf[r, pl.ds(j * 16, 16)] = jnp.zeros((16,), _f32)
            return carry

        lax.fori_loop(0, 32, zrow, 0)
        tbase = s * RPT

        def zc(k, carry):
            pltpu.sync_copy(zbuf, acc.at[pl.ds(tbase + k * 32, 32)])
            return carry

        lax.fori_loop(0, RPT // 32, zc, 0)
        pltpu.sync_copy(zbuf.at[pl.ds(0, RPT % 32)],
                        acc.at[pl.ds(tbase + (RPT // 32) * 32, RPT % 32)])
        plsc.subcore_barrier()

        def loads(i, sl):
            row = wid * nch + i
            pltpu.async_copy(p1.at[pl.ds(row * c_, c_)], pb.at[sl], semL[sl])
            pltpu.async_copy(p2.at[pl.ds(row * c_, c_)], qb.at[sl], semL[sl])
            pltpu.async_copy(idxd.at[row], iv.at[sl], semL[sl])
            pltpu.async_copy(idx2.at[row], iv2.at[sl], semL[sl])

        def drainL(i, sl):
            row = wid * nch + i
            pltpu.make_async_copy(p1.at[pl.ds(row * c_, c_)], pb.at[sl],
                                  semL[sl]).wait()
            pltpu.make_async_copy(p2.at[pl.ds(row * c_, c_)], qb.at[sl],
                                  semL[sl]).wait()
            pltpu.make_async_copy(idxd.at[row], iv.at[sl], semL[sl]).wait()
            pltpu.make_async_copy(idx2.at[row], iv2.at[sl], semL[sl]).wait()

        def fire(sl):
            pltpu.async_copy(pb.at[sl], acc.at[iv.at[sl]], semA[sl], add=True)
            pltpu.async_copy(qb.at[sl], acc.at[iv2.at[sl]], semA[sl], add=True)

        def drainA(sl):
            pltpu.make_async_copy(pb.at[sl], acc.at[iv.at[sl]], semA[sl]).wait()
            pltpu.make_async_copy(qb.at[sl], acc.at[iv2.at[sl]], semA[sl]).wait()

        loads(0, 0)
        loads(1, 1)

        def body2(k, carry):
            i0 = 2 * k
            drainL(i0, 0)
            fire(0)
            drainL(i0 + 1, 1)
            fire(1)
            drainA(0)

            @pl.when(i0 + 2 < nch)
            def _():
                loads(i0 + 2, 0)

            drainA(1)

            @pl.when(i0 + 3 < nch)
            def _():
                loads(i0 + 3, 1)

            return carry

        lax.fori_loop(0, nch // 2, body2, 0)
        drainL(nch - 1, 0)
        fire(0)
        drainA(0)
        plsc.subcore_barrier()
        pltpu.sync_copy(acc.at[pl.ds(tbase, RPT)],
                        out.at[pl.ds(c * NPP + tbase, RPT)])

    return pl.kernel(
        body,
        out_type=jax.ShapeDtypeStruct((NC * NPP, D), _f32),
        mesh=plsc.VectorSubcoreMesh(
            core_axis_name="c", subcore_axis_name="s",
            num_cores=NC, num_subcores=NS,
        ),
        scratch_types=[
            pltpu.VMEM_SHARED((NPP, D), _f32),
            pltpu.VMEM((2, c_, D), _f32),
            pltpu.VMEM((2, c_, D), _f32),
            pltpu.VMEM((2, c_), jnp.int32),
            pltpu.VMEM((2, c_), jnp.int32),
            pltpu.VMEM((32, D), _f32),
            pltpu.SemaphoreType.DMA,
            pltpu.SemaphoreType.DMA,
            pltpu.SemaphoreType.DMA,
            pltpu.SemaphoreType.DMA,
        ],
    )


_sc_scatter_h = _make_sc_scatter(E2, C2)


def _fe_body(we, we1e, be, be1, fe, fb):
    for l in range(3):
        w1e = we1e[l]
        fe[l] = jnp.dot(we[...], w1e, preferred_element_type=_f32)
        fb[l] = jnp.dot(be[...], w1e, preferred_element_type=_f32) + be1[l][None, :]


_fe_call = pl.pallas_call(
    _fe_body,
    out_shape=(
        jax.ShapeDtypeStruct((3, NRBF, D), _f32),
        jax.ShapeDtypeStruct((3, 1, D), _f32),
    ),
)


def _init_body(nf_ref, pos_ref, wn, bn, wda, wsb, h_ref, td_ref, ts_ref):
    nf = nf_ref[...]
    nf = jnp.concatenate([nf[:, :6], nf[:, 6:7] * (1.0 / RES_SCALE)], axis=1)
    h = jnp.dot(nf, wn[...], preferred_element_type=_f32) + bn[...]
    h_ref[...] = h
    a = jnp.dot(h, wda[...], preferred_element_type=_f32)
    b = jnp.dot(h, wsb[...], preferred_element_type=_f32)
    p = pos_ref[...]
    z = jnp.zeros((BN, WG - D - 3), _f32)
    td_ref[...] = jnp.concatenate([a, p, z], axis=1)
    ts_ref[...] = jnp.concatenate([b, -p, z], axis=1)


_init_call = pl.pallas_call(
    _init_body,
    grid=(N // BN,),
    in_specs=[
        pl.BlockSpec((BN, 7), lambda i: (i, 0)),
        pl.BlockSpec((BN, 3), lambda i: (i, 0)),
        pl.BlockSpec((7, D), lambda i: (0, 0)),
        pl.BlockSpec((1, D), lambda i: (0, 0)),
        pl.BlockSpec((D, D), lambda i: (0, 0)),
        pl.BlockSpec((D, D), lambda i: (0, 0)),
    ],
    out_specs=[
        pl.BlockSpec((BN, D), lambda i: (i, 0)),
        pl.BlockSpec((BN, WG), lambda i: (i, 0)),
        pl.BlockSpec((BN, WG), lambda i: (i, 0)),
    ],
    out_shape=[
        jax.ShapeDtypeStruct((N, D), _f32),
        jax.ShapeDtypeStruct((N, WG), _f32),
        jax.ShapeDtypeStruct((N, WG), _f32),
    ],
)


def _edge_body(g_ref, ea_ref, dst_ref, fe, fb, wd2, we2, be2, wx1, bx1, wx2t,
               bx2, p1_ref, p2_ref):
    x = g_ref[...]
    gsum = x[:, :D]
    rel = x[:, D:D + 3]
    d2 = jnp.sum(rel * rel, axis=1, keepdims=True)
    dd = ea_ref[...]
    cen = (lax.broadcasted_iota(jnp.int32, (1, NRBF), 1).astype(_f32)
           * (RMAX / (NRBF - 1)))
    rbf = jnp.exp(-GAMMA * (dd - cen) ** 2)
    pre = (gsum + d2 * wd2[...]
           + jnp.dot(rbf, fe[...], preferred_element_type=_f32) + fb[...])
    m = _silu(pre)
    m = _silu(jnp.dot(m, we2[...], preferred_element_type=_f32) + be2[...])
    t = _silu(jnp.dot(m, wx1[...], preferred_element_type=_f32) + bx1[...])
    w = jnp.sum(t * wx2t[...], axis=1, keepdims=True) + bx2[...]
    p1_ref[...] = m
    rw = rel * w
    be = dst_ref.shape[0]
    dm = lax.rem(dst_ref[...], jnp.full((be, 1), 32, jnp.int32))
    lane = lax.broadcasted_iota(jnp.int32, (1, D), 1)
    lm = lax.rem(lane, jnp.full((1, D), 4, jnp.int32))
    grp = lax.div(lane, jnp.full((1, D), 4, jnp.int32))
    vals = (rw[:, 0:1] * (lm == 0).astype(_f32)
            + rw[:, 1:2] * (lm == 1).astype(_f32)
            + rw[:, 2:3] * (lm == 2).astype(_f32)
            + (lm == 3).astype(_f32))
    p2_ref[...] = jnp.where(grp == dm, vals, 0.0)


def _make_edge_call(ne, be):
    return pl.pallas_call(
        _edge_body,
        grid=(ne // be,),
        in_specs=[
            pl.BlockSpec((be, WO), lambda i: (i, 0)),
            pl.BlockSpec((be, 1), lambda i: (i, 0)),
            pl.BlockSpec((be, 1), lambda i: (i, 0)),
            pl.BlockSpec((NRBF, D), lambda i: (0, 0)),
            pl.BlockSpec((1, D), lambda i: (0, 0)),
            pl.BlockSpec((1, D), lambda i: (0, 0)),
            pl.BlockSpec((D, D), lambda i: (0, 0)),
            pl.BlockSpec((1, D), lambda i: (0, 0)),
            pl.BlockSpec((D, D), lambda i: (0, 0)),
            pl.BlockSpec((1, D), lambda i: (0, 0)),
            pl.BlockSpec((1, D), lambda i: (0, 0)),
            pl.BlockSpec((1, 1), lambda i: (0, 0)),
        ],
        out_specs=[
            pl.BlockSpec((be, D), lambda i: (i, 0)),
            pl.BlockSpec((be, D), lambda i: (i, 0)),
        ],
        out_shape=[
            jax.ShapeDtypeStruct((ne, D), _f32),
            jax.ShapeDtypeStruct((ne, D), _f32),
        ],
    )


_edge_call_h = _make_edge_call(E2, 640)


def _node_body(*args, with_tables):
    nacc = 2 * NHLF
    aref = args[:nacc]
    pdref = args[nacc:2 * nacc]
    (h_ref, pos_ref, wh1a, wh1b, bh1, wh2, bh2, lg, lb) = args[2 * nacc:2 * nacc + 9]
    rest = args[2 * nacc + 9:]
    if with_tables:
        wda, wsb, hn_ref, pn_ref, td_ref, ts_ref = rest
    else:
        hn_ref, pn_ref = rest
    agg = functools.reduce(lambda a, b: a + b, [r[...] for r in aref])
    pacc = functools.reduce(lambda a, b: a + b, [r[...] for r in pdref])
    posd = pacc[:, :3]
    deg = pacc[:, 3:4]
    pn = pos_ref[...] + posd / (deg + 1.0)
    hh = h_ref[...]
    u = _silu(jnp.dot(hh, wh1a[...], preferred_element_type=_f32)
              + jnp.dot(agg, wh1b[...], preferred_element_type=_f32) + bh1[...])
    hn = hh + jnp.dot(u, wh2[...], preferred_element_type=_f32) + bh2[...]
    mu = jnp.mean(hn, axis=1, keepdims=True)
    var = jnp.mean((hn - mu) ** 2, axis=1, keepdims=True)
    hn = (hn - mu) * lax.rsqrt(var + 1e-5) * lg[...] + lb[...]
    hn_ref[...] = hn
    pn_ref[...] = pn
    if with_tables:
        a = jnp.dot(hn, wda[...], preferred_element_type=_f32)
        b = jnp.dot(hn, wsb[...], preferred_element_type=_f32)
        z = jnp.zeros((BN, WG - D - 3), _f32)
        td_ref[...] = jnp.concatenate([a, pn, z], axis=1)
        ts_ref[...] = jnp.concatenate([b, -pn, z], axis=1)


def _make_node_call(with_tables):
    n_extra_in = 2 if with_tables else 0
    out_shapes = [
        jax.ShapeDtypeStruct((N, D), _f32),
        jax.ShapeDtypeStruct((N, 3), _f32),
    ]
    out_specs = [
        pl.BlockSpec((BN, D), lambda i: (i, 0)),
        pl.BlockSpec((BN, 3), lambda i: (i, 0)),
    ]
    if with_tables:
        out_shapes += [jax.ShapeDtypeStruct((N, WG), _f32)] * 2
        out_specs += [pl.BlockSpec((BN, WG), lambda i: (i, 0))] * 2
    return pl.pallas_call(
        functools.partial(_node_body, with_tables=with_tables),
        grid=(N // BN,),
        in_specs=[pl.BlockSpec((BN, D), lambda i: (i, 0))] * (2 * NHLF)
        + [pl.BlockSpec((BN, 4), lambda i: (i, 0))] * (2 * NHLF)
        + [
            pl.BlockSpec((BN, D), lambda i: (i, 0)),
            pl.BlockSpec((BN, 3), lambda i: (i, 0)),
            pl.BlockSpec((D, D), lambda i: (0, 0)),
            pl.BlockSpec((D, D), lambda i: (0, 0)),
            pl.BlockSpec((1, D), lambda i: (0, 0)),
            pl.BlockSpec((D, D), lambda i: (0, 0)),
            pl.BlockSpec((1, D), lambda i: (0, 0)),
            pl.BlockSpec((1, D), lambda i: (0, 0)),
            pl.BlockSpec((1, D), lambda i: (0, 0)),
        ] + [pl.BlockSpec((D, D), lambda i: (0, 0))] * n_extra_in,
        out_specs=out_specs,
        out_shape=out_shapes,
    )


_node_mid = _make_node_call(True)
_node_last = _make_node_call(False)


def kernel(node_feat, edge_attr, pos, Wn, bn, We, be, We1, be1, We2, be2,
           Wx1, bx1, Wx2, bx2, Wh1, bh1, Wh2, bh2, ln_g, ln_b, edge_index):
    src = edge_index[0]
    dst = edge_index[1]
    halves = []
    for hf in range(NHLF):
        lo = hf * E2
        d_h = lax.slice_in_dim(dst, lo, lo + E2)
        s_h = lax.slice_in_dim(src, lo, lo + E2)
        halves.append(dict(
            dst=d_h, src=s_h,
            idxd2=d_h.reshape(NW * NCH2, C2),
            idx2=(PB + d_h // 32).reshape(NW * NCH2, C2),
            ea=lax.slice_in_dim(edge_attr, lo, lo + E2),
            dstc=d_h.reshape(E2, 1),
        ))

    fe, fb = _fe_call(We, We1[:, 2 * D + 1:, :], be.reshape(1, D), be1)
    h, td, ts = _init_call(node_feat, pos, Wn, bn.reshape(1, D),
                           We1[0, :D, :], We1[0, D:2 * D, :])
    for l in range(3):
        gs = [_sc_gather_h(td, ts, hv["dst"], hv["src"]) for hv in halves]
        ps = [_edge_call_h(g, hv["ea"], hv["dstc"], fe[l], fb[l],
                           We1[l, 2 * D, :].reshape(1, D), We2[l],
                           be2[l].reshape(1, D), Wx1[l], bx1[l].reshape(1, D),
                           Wx2[l].reshape(1, D), bx2[l].reshape(1, 1))
              for g, hv in zip(gs, halves)]
        outs = [_sc_scatter_h(p1, p2, hv["idxd2"], hv["idx2"])
                for (p1, p2), hv in zip(ps, halves)]
        accs = []
        pds = []
        for outm in outs:
            accs += [outm[:N], outm[NPP:NPP + N]]
            pds += [outm[PB:PB + NPOS].reshape(NPOS * 32, 4)[:N],
                    outm[NPP + PB:NPP + PB + NPOS].reshape(NPOS * 32, 4)[:N]]
        if l < 2:
            h, pos, td, ts = _node_mid(
                *accs, *pds, h, pos,
                Wh1[l, :D, :], Wh1[l, D:, :], bh1[l].reshape(1, D),
                Wh2[l], bh2[l].reshape(1, D),
                ln_g[l].reshape(1, D), ln_b[l].reshape(1, D),
                We1[l + 1, :D, :], We1[l + 1, D:2 * D, :])
        else:
            h, pos = _node_last(
                *accs, *pds, h, pos,
                Wh1[l, :D, :], Wh1[l, D:, :], bh1[l].reshape(1, D),
                Wh2[l], bh2[l].reshape(1, D),
                ln_g[l].reshape(1, D), ln_b[l].reshape(1, D))
    return h, pos


# speedup vs baseline: 3.3024x; 1.0119x over previous
import functools

import jax
import jax.numpy as jnp
from jax import lax
from jax.experimental import pallas as pl
from jax.experimental.pallas import tpu as pltpu
from jax.experimental.pallas import tpu_sc as plsc

N = 10000
E = 320000
D = 128
WG = 256
WO = 144
NRBF = 32
RMAX = 10.0
GAMMA = 1.0 / ((RMAX / NRBF) ** 2)
RES_SCALE = 1000.0

NC = 2
NS = 16
NW = NC * NS
EPW = E // NW
C = 80
NCH = EPW // C
NHLF = 2
E2 = E // NHLF
EPW2 = E2 // NW
C2 = 40
NCH2 = EPW2 // C2
PB = 10000
NPOS = 320
NPP = 10368
RPT = NPP // NS

BE = 512
BN = 1000

_f32 = jnp.float32


def _silu(x):
    return x * jax.nn.sigmoid(x)


def _make_sc_gather(ne, c_):
    epw = ne // NW
    nch = epw // c_

    def body(td, ts, idxd, idxs, g, idb, isb, bufd, bufs, obuf,
             semd0, sems0, semd1, sems1, semw0):
        c = lax.axis_index("c")
        s = lax.axis_index("s")
        wid = c * NS + s
        sems = ((semd0, sems0), (semd1, sems1))

        pltpu.sync_copy(idxd.at[pl.ds(wid * epw, epw)], idb)
        pltpu.sync_copy(idxs.at[pl.ds(wid * epw, epw)], isb)

        def start(i, sl):
            pltpu.async_copy(td.at[idb.at[pl.ds(i * c_, c_)]], bufd.at[sl],
                             sems[sl][0])
            pltpu.async_copy(ts.at[isb.at[pl.ds(i * c_, c_)]], bufs.at[sl],
                             sems[sl][1])

        def finish(i, sl):
            row = wid * nch + i
            pltpu.make_async_copy(td.at[idb.at[pl.ds(i * c_, c_)]], bufd.at[sl],
                                  sems[sl][0]).wait()
            pltpu.make_async_copy(ts.at[isb.at[pl.ds(i * c_, c_)]], bufs.at[sl],
                                  sems[sl][1]).wait()

            @pl.when(i >= 1)
            def _():
                pltpu.make_async_copy(obuf, g.at[pl.ds(row * c_, c_)],
                                      semw0).wait()

            def add_row(r, carry):
                for j in range(WO // 16):
                    obuf[r, pl.ds(j * 16, 16)] = (
                        bufd[sl, r, pl.ds(j * 16, 16)]
                        + bufs[sl, r, pl.ds(j * 16, 16)])
                return carry

            lax.fori_loop(0, c_, add_row, 0)
            pltpu.async_copy(obuf, g.at[pl.ds(row * c_, c_)], semw0)

        start(0, 0)

        def body2(k, carry):
            i0 = 2 * k
            start(i0 + 1, 1)
            finish(i0, 0)
            start(i0 + 2, 0)
            finish(i0 + 1, 1)
            return carry

        lax.fori_loop(0, (nch - 1) // 2, body2, 0)
        finish(nch - 1, 0)
        pltpu.make_async_copy(obuf, g.at[pl.ds(0, c_)], semw0).wait()

    return pl.kernel(
        body,
        out_type=jax.ShapeDtypeStruct((ne, WO), _f32),
        mesh=plsc.VectorSubcoreMesh(
            core_axis_name="c", subcore_axis_name="s",
            num_cores=NC, num_subcores=NS,
        ),
        scratch_types=[
            pltpu.VMEM((epw,), jnp.int32),
            pltpu.VMEM((epw,), jnp.int32),
            pltpu.VMEM((2, c_, WG), _f32),
            pltpu.VMEM((2, c_, WG), _f32),
            pltpu.VMEM((c_, WO), _f32),
            pltpu.SemaphoreType.DMA,
            pltpu.SemaphoreType.DMA,
            pltpu.SemaphoreType.DMA,
            pltpu.SemaphoreType.DMA,
            pltpu.SemaphoreType.DMA,
        ],
    )


_sc_gather_h = _make_sc_gather(E2, C2)


def _make_sc_scatter(ne, c_):
    epw = ne // NW
    nch = epw // c_

    def body(p1, p2, idxd, idx2, out, acc, pb, qb, iv, iv2, zbuf,
             semA0, semA1, semL0, semL1):
        c = lax.axis_index("c")
        s = lax.axis_index("s")
        wid = c * NS + s
        semA = (semA0, semA1)
        semL = (semL0, semL1)

        def zrow(r, carry):
            for j in range(D // 16):
                zbuf[r, pl.ds(j * 16, 16)] = jnp.zeros((16,), _f32)
            return carry

        lax.fori_loop(0, 32, zrow, 0)
        tbase = s * RPT

        def zc(k, carry):
            pltpu.sync_copy(zbuf, acc.at[pl.ds(tbase + k * 32, 32)])
            return carry

        lax.fori_loop(0, RPT // 32, zc, 0)
        pltpu.sync_copy(zbuf.at[pl.ds(0, RPT % 32)],
                        acc.at[pl.ds(tbase + (RPT // 32) * 32, RPT % 32)])
        plsc.subcore_barrier()

        def loads(i, sl):
            row = wid * nch + i
            pltpu.async_copy(p1.at[pl.ds(row * c_, c_)], pb.at[sl], semL[sl])
            pltpu.async_copy(p2.at[pl.ds(row * c_, c_)], qb.at[sl], semL[sl])
            pltpu.async_copy(idxd.at[row], iv.at[sl], semL[sl])
            pltpu.async_copy(idx2.at[row], iv2.at[sl], semL[sl])

        def drainL(i, sl):
            row = wid * nch + i
            pltpu.make_async_copy(p1.at[pl.ds(row * c_, c_)], pb.at[sl],
                                  semL[sl]).wait()
            pltpu.make_async_copy(p2.at[pl.ds(row * c_, c_)], qb.at[sl],
                                  semL[sl]).wait()
            pltpu.make_async_copy(idxd.at[row], iv.at[sl], semL[sl]).wait()
            pltpu.make_async_copy(idx2.at[row], iv2.at[sl], semL[sl]).wait()

        def fire(sl):
            pltpu.async_copy(pb.at[sl], acc.at[iv.at[sl]], semA[sl], add=True)
            pltpu.async_copy(qb.at[sl], acc.at[iv2.at[sl]], semA[sl], add=True)

        def drainA(sl):
            pltpu.make_async_copy(pb.at[sl], acc.at[iv.at[sl]], semA[sl]).wait()
            pltpu.make_async_copy(qb.at[sl], acc.at[iv2.at[sl]], semA[sl]).wait()

        loads(0, 0)
        loads(1, 1)

        def body2(k, carry):
            i0 = 2 * k
            drainL(i0, 0)
            fire(0)
            drainL(i0 + 1, 1)
            fire(1)
            drainA(0)

            @pl.when(i0 + 2 < nch)
            def _():
                loads(i0 + 2, 0)

            drainA(1)

            @pl.when(i0 + 3 < nch)
            def _():
                loads(i0 + 3, 1)

            return carry

        lax.fori_loop(0, nch // 2, body2, 0)
        drainL(nch - 1, 0)
        fire(0)
        drainA(0)
        plsc.subcore_barrier()
        pltpu.sync_copy(acc.at[pl.ds(tbase, RPT)],
                        out.at[pl.ds(c * NPP + tbase, RPT)])

    return pl.kernel(
        body,
        out_type=jax.ShapeDtypeStruct((NC * NPP, D), _f32),
        mesh=plsc.VectorSubcoreMesh(
            core_axis_name="c", subcore_axis_name="s",
            num_cores=NC, num_subcores=NS,
        ),
        scratch_types=[
            pltpu.VMEM_SHARED((NPP, D), _f32),
            pltpu.VMEM((2, c_, D), _f32),
            pltpu.VMEM((2, c_, D), _f32),
            pltpu.VMEM((2, c_), jnp.int32),
            pltpu.VMEM((2, c_), jnp.int32),
            pltpu.VMEM((32, D), _f32),
            pltpu.SemaphoreType.DMA,
            pltpu.SemaphoreType.DMA,
            pltpu.SemaphoreType.DMA,
            pltpu.SemaphoreType.DMA,
        ],
    )


_sc_scatter_h = _make_sc_scatter(E2, C2)


def _fe_body(we, we1e, be, be1, fe, fb):
    for l in range(3):
        w1e = we1e[l]
        fe[l] = jnp.dot(we[...], w1e, preferred_element_type=_f32)
        fb[l] = jnp.dot(be[...], w1e, preferred_element_type=_f32) + be1[l][None, :]


_fe_call = pl.pallas_call(
    _fe_body,
    out_shape=(
        jax.ShapeDtypeStruct((3, NRBF, D), _f32),
        jax.ShapeDtypeStruct((3, 1, D), _f32),
    ),
)


def _init_body(nf_ref, pos_ref, wn, bn, wda, wsb, h_ref, td_ref, ts_ref):
    nf = nf_ref[...]
    nf = jnp.concatenate([nf[:, :6], nf[:, 6:7] * (1.0 / RES_SCALE)], axis=1)
    h = jnp.dot(nf, wn[...], preferred_element_type=_f32) + bn[...]
    h_ref[...] = h
    a = jnp.dot(h, wda[...], preferred_element_type=_f32)
    b = jnp.dot(h, wsb[...], preferred_element_type=_f32)
    p = pos_ref[...]
    z = jnp.zeros((BN, WG - D - 3), _f32)
    td_ref[...] = jnp.concatenate([a, p, z], axis=1)
    ts_ref[...] = jnp.concatenate([b, -p, z], axis=1)


_init_call = pl.pallas_call(
    _init_body,
    grid=(N // BN,),
    in_specs=[
        pl.BlockSpec((BN, 7), lambda i: (i, 0)),
        pl.BlockSpec((BN, 3), lambda i: (i, 0)),
        pl.BlockSpec((7, D), lambda i: (0, 0)),
        pl.BlockSpec((1, D), lambda i: (0, 0)),
        pl.BlockSpec((D, D), lambda i: (0, 0)),
        pl.BlockSpec((D, D), lambda i: (0, 0)),
    ],
    out_specs=[
        pl.BlockSpec((BN, D), lambda i: (i, 0)),
        pl.BlockSpec((BN, WG), lambda i: (i, 0)),
        pl.BlockSpec((BN, WG), lambda i: (i, 0)),
    ],
    out_shape=[
        jax.ShapeDtypeStruct((N, D), _f32),
        jax.ShapeDtypeStruct((N, WG), _f32),
        jax.ShapeDtypeStruct((N, WG), _f32),
    ],
)


def _edge_body(g_ref, ea_ref, dst_ref, fe, fb, wd2, we2, be2, wx1, bx1, wx2t,
               bx2, p1_ref, p2_ref):
    x = g_ref[...]
    gsum = x[:, :D]
    rel = x[:, D:D + 3]
    d2 = jnp.sum(rel * rel, axis=1, keepdims=True)
    dd = ea_ref[...]
    cen = (lax.broadcasted_iota(jnp.int32, (1, NRBF), 1).astype(_f32)
           * (RMAX / (NRBF - 1)))
    rbf = jnp.exp(-GAMMA * (dd - cen) ** 2)
    pre = (gsum + d2 * wd2[...]
           + jnp.dot(rbf, fe[...], preferred_element_type=_f32) + fb[...])
    m = _silu(pre)
    m = _silu(jnp.dot(m, we2[...], preferred_element_type=_f32) + be2[...])
    t = _silu(jnp.dot(m, wx1[...], preferred_element_type=_f32) + bx1[...])
    w = jnp.sum(t * wx2t[...], axis=1, keepdims=True) + bx2[...]
    p1_ref[...] = m
    rw = rel * w
    be = dst_ref.shape[0]
    dm = lax.rem(dst_ref[...], jnp.full((be, 1), 32, jnp.int32))
    lane = lax.broadcasted_iota(jnp.int32, (1, D), 1)
    lm = lax.rem(lane, jnp.full((1, D), 4, jnp.int32))
    grp = lax.div(lane, jnp.full((1, D), 4, jnp.int32))
    vals = (rw[:, 0:1] * (lm == 0).astype(_f32)
            + rw[:, 1:2] * (lm == 1).astype(_f32)
            + rw[:, 2:3] * (lm == 2).astype(_f32)
            + (lm == 3).astype(_f32))
    p2_ref[...] = jnp.where(grp == dm, vals, 0.0)


def _make_edge_call(ne, be):
    return pl.pallas_call(
        _edge_body,
        grid=(ne // be,),
        in_specs=[
            pl.BlockSpec((be, WO), lambda i: (i, 0)),
            pl.BlockSpec((be, 1), lambda i: (i, 0)),
            pl.BlockSpec((be, 1), lambda i: (i, 0)),
            pl.BlockSpec((NRBF, D), lambda i: (0, 0)),
            pl.BlockSpec((1, D), lambda i: (0, 0)),
            pl.BlockSpec((1, D), lambda i: (0, 0)),
            pl.BlockSpec((D, D), lambda i: (0, 0)),
            pl.BlockSpec((1, D), lambda i: (0, 0)),
            pl.BlockSpec((D, D), lambda i: (0, 0)),
            pl.BlockSpec((1, D), lambda i: (0, 0)),
            pl.BlockSpec((1, D), lambda i: (0, 0)),
            pl.BlockSpec((1, 1), lambda i: (0, 0)),
        ],
        out_specs=[
            pl.BlockSpec((be, D), lambda i: (i, 0)),
            pl.BlockSpec((be, D), lambda i: (i, 0)),
        ],
        out_shape=[
            jax.ShapeDtypeStruct((ne, D), _f32),
            jax.ShapeDtypeStruct((ne, D), _f32),
        ],
    )


_edge_call_h = _make_edge_call(E2, 1280)


def _node_body(*args, with_tables):
    nacc = 2 * NHLF
    aref = args[:nacc]
    pdref = args[nacc:2 * nacc]
    (h_ref, pos_ref, wh1a, wh1b, bh1, wh2, bh2, lg, lb) = args[2 * nacc:2 * nacc + 9]
    rest = args[2 * nacc + 9:]
    if with_tables:
        wda, wsb, hn_ref, pn_ref, td_ref, ts_ref = rest
    else:
        hn_ref, pn_ref = rest
    agg = functools.reduce(lambda a, b: a + b, [r[...] for r in aref])
    pacc = functools.reduce(lambda a, b: a + b, [r[...] for r in pdref])
    posd = pacc[:, :3]
    deg = pacc[:, 3:4]
    pn = pos_ref[...] + posd / (deg + 1.0)
    hh = h_ref[...]
    u = _silu(jnp.dot(hh, wh1a[...], preferred_element_type=_f32)
              + jnp.dot(agg, wh1b[...], preferred_element_type=_f32) + bh1[...])
    hn = hh + jnp.dot(u, wh2[...], preferred_element_type=_f32) + bh2[...]
    mu = jnp.mean(hn, axis=1, keepdims=True)
    var = jnp.mean((hn - mu) ** 2, axis=1, keepdims=True)
    hn = (hn - mu) * lax.rsqrt(var + 1e-5) * lg[...] + lb[...]
    hn_ref[...] = hn
    pn_ref[...] = pn
    if with_tables:
        a = jnp.dot(hn, wda[...], preferred_element_type=_f32)
        b = jnp.dot(hn, wsb[...], preferred_element_type=_f32)
        z = jnp.zeros((BN, WG - D - 3), _f32)
        td_ref[...] = jnp.concatenate([a, pn, z], axis=1)
        ts_ref[...] = jnp.concatenate([b, -pn, z], axis=1)


def _make_node_call(with_tables):
    n_extra_in = 2 if with_tables else 0
    out_shapes = [
        jax.ShapeDtypeStruct((N, D), _f32),
        jax.ShapeDtypeStruct((N, 3), _f32),
    ]
    out_specs = [
        pl.BlockSpec((BN, D), lambda i: (i, 0)),
        pl.BlockSpec((BN, 3), lambda i: (i, 0)),
    ]
    if with_tables:
        out_shapes += [jax.ShapeDtypeStruct((N, WG), _f32)] * 2
        out_specs += [pl.BlockSpec((BN, WG), lambda i: (i, 0))] * 2
    return pl.pallas_call(
        functools.partial(_node_body, with_tables=with_tables),
        grid=(N // BN,),
        in_specs=[pl.BlockSpec((BN, D), lambda i: (i, 0))] * (2 * NHLF)
        + [pl.BlockSpec((BN, 4), lambda i: (i, 0))] * (2 * NHLF)
        + [
            pl.BlockSpec((BN, D), lambda i: (i, 0)),
            pl.BlockSpec((BN, 3), lambda i: (i, 0)),
            pl.BlockSpec((D, D), lambda i: (0, 0)),
            pl.BlockSpec((D, D), lambda i: (0, 0)),
            pl.BlockSpec((1, D), lambda i: (0, 0)),
            pl.BlockSpec((D, D), lambda i: (0, 0)),
            pl.BlockSpec((1, D), lambda i: (0, 0)),
            pl.BlockSpec((1, D), lambda i: (0, 0)),
            pl.BlockSpec((1, D), lambda i: (0, 0)),
        ] + [pl.BlockSpec((D, D), lambda i: (0, 0))] * n_extra_in,
        out_specs=out_specs,
        out_shape=out_shapes,
    )


_node_mid = _make_node_call(True)
_node_last = _make_node_call(False)


def kernel(node_feat, edge_attr, pos, Wn, bn, We, be, We1, be1, We2, be2,
           Wx1, bx1, Wx2, bx2, Wh1, bh1, Wh2, bh2, ln_g, ln_b, edge_index):
    src = edge_index[0]
    dst = edge_index[1]
    halves = []
    for hf in range(NHLF):
        lo = hf * E2
        d_h = lax.slice_in_dim(dst, lo, lo + E2)
        s_h = lax.slice_in_dim(src, lo, lo + E2)
        halves.append(dict(
            dst=d_h, src=s_h,
            idxd2=d_h.reshape(NW * NCH2, C2),
            idx2=(PB + d_h // 32).reshape(NW * NCH2, C2),
            ea=lax.slice_in_dim(edge_attr, lo, lo + E2),
            dstc=d_h.reshape(E2, 1),
        ))

    fe, fb = _fe_call(We, We1[:, 2 * D + 1:, :], be.reshape(1, D), be1)
    h, td, ts = _init_call(node_feat, pos, Wn, bn.reshape(1, D),
                           We1[0, :D, :], We1[0, D:2 * D, :])
    for l in range(3):
        gs = [_sc_gather_h(td, ts, hv["dst"], hv["src"]) for hv in halves]
        ps = [_edge_call_h(g, hv["ea"], hv["dstc"], fe[l], fb[l],
                           We1[l, 2 * D, :].reshape(1, D), We2[l],
                           be2[l].reshape(1, D), Wx1[l], bx1[l].reshape(1, D),
                           Wx2[l].reshape(1, D), bx2[l].reshape(1, 1))
              for g, hv in zip(gs, halves)]
        outs = [_sc_scatter_h(p1, p2, hv["idxd2"], hv["idx2"])
                for (p1, p2), hv in zip(ps, halves)]
        accs = []
        pds = []
        for outm in outs:
            accs += [outm[:N], outm[NPP:NPP + N]]
            pds += [outm[PB:PB + NPOS].reshape(NPOS * 32, 4)[:N],
                    outm[NPP + PB:NPP + PB + NPOS].reshape(NPOS * 32, 4)[:N]]
        if l < 2:
            h, pos, td, ts = _node_mid(
                *accs, *pds, h, pos,
                Wh1[l, :D, :], Wh1[l, D:, :], bh1[l].reshape(1, D),
                Wh2[l], bh2[l].reshape(1, D),
                ln_g[l].reshape(1, D), ln_b[l].reshape(1, D),
                We1[l + 1, :D, :], We1[l + 1, D:2 * D, :])
        else:
            h, pos = _node_last(
                *accs, *pds, h, pos,
                Wh1[l, :D, :], Wh1[l, D:, :], bh1[l].reshape(1, D),
                Wh2[l], bh2[l].reshape(1, D),
                ln_g[l].reshape(1, D), ln_b[l].reshape(1, D))
    return h, pos
